# Initial kernel scaffold; baseline (speedup 1.0000x reference)
import jax
import jax.numpy as jnp
from jax.experimental import pallas as pl


def kernel(x, params, edge_feat, src_ids, dst_ids):
    raise NotImplementedError("write your pallas kernel here")


# speedup vs baseline: 16.6947x; 16.6947x over previous
import functools

import jax
import jax.numpy as jnp
from jax import lax
from jax.experimental import pallas as pl
from jax.experimental.pallas import tpu as pltpu
from jax.experimental.pallas import tpu_sc as plsc

_N = 10000
_E = 160000
_NDIM = 128
_H = 8
_DH = 16
_EDIM = 16
_L = 2
_R = 128

_NC = 2
_NS = 16
_NW = _NC * _NS

_CH = 128
_NCHUNK = 40
_EW = _CH * _NCHUNK
_EPAD = _EW * _NW

_SW = 384
_DW = 256

_NPAD = 10240
_CHS = 128
_NCHUNK_S = _EW // _CHS

_f32 = jnp.float32


def _make_gather():
    widths = [_SW, _DW]
    nt = 2
    mesh = plsc.VectorSubcoreMesh(core_axis_name="c", subcore_axis_name="s")
    out_type = [jax.ShapeDtypeStruct((_EPAD, w), _f32) for w in widths]
    scratch = (
        [pltpu.VMEM((_CH,), jnp.int32) for _ in widths]
        + [pltpu.VMEM((_CH, w), _f32) for w in widths]
        + [pltpu.SemaphoreType.DMA for _ in widths]
    )

    @functools.partial(pl.kernel, out_type=out_type, mesh=mesh,
                       scratch_types=scratch)
    def gather_kernel(*refs):
        tables = refs[0:nt]
        idxs = refs[nt:2 * nt]
        outs = refs[2 * nt:3 * nt]
        ibufs = refs[3 * nt:4 * nt]
        rbufs = refs[4 * nt:5 * nt]
        sems = refs[5 * nt:6 * nt]
        wid = lax.axis_index("s") * _NC + lax.axis_index("c")
        base = pl.multiple_of(wid * _EW, 8)

        def body(i, carry):
            off = pl.multiple_of(base + i * _CH, 8)
            for t in range(nt):
                pltpu.sync_copy(idxs[t].at[pl.ds(off, _CH)], ibufs[t])
            descs = [
                pltpu.async_copy(tables[t].at[ibufs[t]], rbufs[t], sems[t])
                for t in range(nt)
            ]
            for d in descs:
                d.wait()
            for t in range(nt):
                pltpu.sync_copy(rbufs[t], outs[t].at[pl.ds(off, _CH)])
            return carry

        lax.fori_loop(0, _NCHUNK, body, 0)

    return gather_kernel


def _make_scatter():
    mesh = plsc.VectorSubcoreMesh(core_axis_name="c", subcore_axis_name="s")
    out_type = jax.ShapeDtypeStruct((_NC, _NPAD, _NDIM), _f32)
    scratch = [
        pltpu.VMEM_SHARED((_NPAD, _NDIM), _f32),
        pltpu.VMEM((_CHS,), jnp.int32),
        pltpu.VMEM((_CHS, _NDIM), _f32),
    ]

    @functools.partial(pl.kernel, out_type=out_type, mesh=mesh,
                       scratch_types=scratch)
    def scatter_kernel(w_hbm, dst_hbm, oz_hbm, o_out, o_sh, ib, wb):
        c = lax.axis_index("c")
        s = lax.axis_index("s")

        @pl.when(s == 0)
        def _init():
            pltpu.sync_copy(oz_hbm, o_sh)

        plsc.subcore_barrier()

        base = pl.multiple_of((c * _NS + s) * _EW, 8)

        def body(i, carry):
            off = pl.multiple_of(base + i * _CHS, 8)
            pltpu.sync_copy(dst_hbm.at[pl.ds(off, _CHS)], ib)
            pltpu.sync_copy(w_hbm.at[pl.ds(off, _CHS)], wb)
            pltpu.sync_copy(wb, o_sh.at[ib], add=True)
            return carry

        lax.fori_loop(0, _NCHUNK_S, body, 0)
        plsc.subcore_barrier()

        @pl.when(s == 0)
        def _writeout():
            pltpu.sync_copy(o_sh, o_out.at[c])

    return scatter_kernel


_BN = 1000
_BE = 2048


def _proj_body(x_ref, ws, bs, wd, bd, st_ref, dt_ref):
    x = x_ref[...]
    st_ref[...] = jnp.dot(x, ws[...], preferred_element_type=_f32) + bs[...]
    dt_ref[...] = jnp.dot(x, wd[...], preferred_element_type=_f32) + bd[...]


def _proj_call(x, ws, bs, wd, bd):
    grid = (_N // _BN,)
    full = lambda shape: pl.BlockSpec(shape, lambda i: (0,) * len(shape))
    rows = lambda w: pl.BlockSpec((_BN, w), lambda i: (i, 0))
    return pl.pallas_call(
        _proj_body,
        grid=grid,
        in_specs=[rows(_NDIM), full((_NDIM, _SW)), full((1, _SW)),
                  full((_NDIM, _DW)), full((1, _DW))],
        out_specs=[rows(_SW), rows(_DW)],
        out_shape=[jax.ShapeDtypeStruct((_N, _SW), _f32),
                   jax.ShapeDtypeStruct((_N, _DW), _f32)],
    )(x, ws, bs, wd, bd)


def _ln_mm(x, ones_mm, g, b):
    m = jnp.dot(x, ones_mm, preferred_element_type=_f32)
    v = jnp.dot(x * x, ones_mm, preferred_element_type=_f32) - m * m
    return (x - m) / jnp.sqrt(v + 1e-5) * g + b


def _edge_body(use_feat, ks, vs, xss, qd, xdd, lg_in, rel,
               wlg, ba, ge1, be1, w1, b1, w2, b2, ge2, be2,
               w_ref, e128_ref, lgn_ref):
    r16 = lax.broadcasted_iota(jnp.int32, (_EDIM, _NDIM), 0)
    c16 = lax.broadcasted_iota(jnp.int32, (_EDIM, _NDIM), 1)
    t16 = (c16 % _EDIM == r16).astype(_f32)
    rs = lax.broadcasted_iota(jnp.int32, (_NDIM, _H), 0)
    cs = lax.broadcasted_iota(jnp.int32, (_NDIM, _H), 1)
    sel = (rs // _DH == cs).astype(_f32)
    rb = lax.broadcasted_iota(jnp.int32, (_H, _NDIM), 0)
    cb = lax.broadcasted_iota(jnp.int32, (_H, _NDIM), 1)
    selt = (cb // _DH == rb).astype(_f32)
    re = lax.broadcasted_iota(jnp.int32, (_NDIM, _EDIM), 0)
    ce = lax.broadcasted_iota(jnp.int32, (_NDIM, _EDIM), 1)
    ext16 = (re == ce).astype(_f32)
    xss = jnp.dot(xss[...], ext16, preferred_element_type=_f32)
    xdd = jnp.dot(xdd[...], ext16, preferred_element_type=_f32)

    if use_feat:
        lanes = lax.broadcasted_iota(jnp.int32, (_BE, _R), 1)
        onehot = (lg_in[...] == lanes).astype(_f32)
        lgb = jnp.dot(onehot, rel[...], preferred_element_type=_f32)
    else:
        lgb = lg_in[...]
    lg128 = jnp.dot(lgb, t16, preferred_element_type=_f32)
    t = qd[...] * (ks[...] + lg128)
    esc = jnp.exp(jnp.dot(t, sel, preferred_element_type=_f32) * 0.25)
    i = pl.program_id(0)
    rid = i * _BE + lax.broadcasted_iota(jnp.int32, (_BE, _H), 0)
    esc = jnp.where(rid < _E, esc, 0.0)
    e128 = jnp.dot(esc, selt, preferred_element_type=_f32)
    w_ref[...] = (vs[...] + lg128) * e128
    e128_ref[...] = e128

    one16 = jnp.full((_EDIM, _EDIM), 1.0 / _EDIM, _f32)
    a = jnp.tanh(jnp.dot(lgb, wlg[...], preferred_element_type=_f32)
                 + xss + xdd + ba[...])
    eh = _ln_mm(lgb + a, one16, ge1[...], be1[...])
    ff = jnp.dot(jnp.maximum(jnp.dot(eh, w1[...], preferred_element_type=_f32)
                             + b1[...], 0.0),
                 w2[...], preferred_element_type=_f32) + b2[...]
    lgn_ref[...] = _ln_mm(eh + ff, one16, ge2[...], be2[...])


def _edge_call(use_feat, srcg, dstg, lg_or_feat, rel,
               wlg, ba, ge1, be1, w1, b1, w2, b2, ge2, be2):
    grid = (_EPAD // _BE,)
    full = lambda shape: pl.BlockSpec(shape, lambda i: (0,) * len(shape))
    rows = lambda w: pl.BlockSpec((_BE, w), lambda i: (i, 0))
    lg_spec = (pl.BlockSpec((_BE, 1), lambda i: (i, 0)) if use_feat
               else rows(_EDIM))
    return pl.pallas_call(
        functools.partial(_edge_body, use_feat),
        grid=grid,
        in_specs=[pl.BlockSpec((_BE, _NDIM), lambda i: (i, 0)),
                  pl.BlockSpec((_BE, _NDIM), lambda i: (i, 1)),
                  pl.BlockSpec((_BE, _NDIM), lambda i: (i, 2)),
                  pl.BlockSpec((_BE, _NDIM), lambda i: (i, 0)),
                  pl.BlockSpec((_BE, _NDIM), lambda i: (i, 1)),
                  lg_spec,
                  full((_R, _EDIM)),
                  full((_EDIM, _EDIM)), full((1, _EDIM)),
                  full((1, _EDIM)), full((1, _EDIM)),
                  full((_EDIM, 4 * _EDIM)), full((1, 4 * _EDIM)),
                  full((4 * _EDIM, _EDIM)), full((1, _EDIM)),
                  full((1, _EDIM)), full((1, _EDIM))],
        out_specs=[rows(_NDIM), rows(_NDIM), rows(_EDIM)],
        out_shape=[jax.ShapeDtypeStruct((_EPAD, _NDIM), _f32),
                   jax.ShapeDtypeStruct((_EPAD, _NDIM), _f32),
                   jax.ShapeDtypeStruct((_EPAD, _EDIM), _f32)],
    )(srcg, srcg, srcg, dstg, dstg, lg_or_feat, rel,
      wlg, ba, ge1, be1, w1, b1, w2, b2, ge2, be2)


def _node_body(x_ref, op, sp, wo, bo, g1, b1, w1, f1, w2, f2, g2, b2,
               out_ref):
    o_un = op[0] + op[1]
    sv = sp[0] + sp[1]
    o = o_un / (sv + 1e-12)
    o = jnp.dot(o, wo[...], preferred_element_type=_f32) + bo[...]
    one128 = jnp.full((_NDIM, _NDIM), 1.0 / _NDIM, _f32)
    h = _ln_mm(x_ref[...] + o, one128, g1[...], b1[...])
    ff = jnp.dot(jnp.maximum(jnp.dot(h, w1[...], preferred_element_type=_f32)
                             + f1[...], 0.0),
                 w2[...], preferred_element_type=_f32) + f2[...]
    out_ref[...] = _ln_mm(h + ff, one128, g2[...], b2[...])


def _node_call(x, o_part, s_part, wo, bo, g1, b1, w1, f1, w2, f2, g2, b2):
    grid = (_N // _BN,)
    full = lambda shape: pl.BlockSpec(shape, lambda i: (0,) * len(shape))
    return pl.pallas_call(
        _node_body,
        grid=grid,
        in_specs=[pl.BlockSpec((_BN, _NDIM), lambda i: (i, 0)),
                  pl.BlockSpec((_NC, _BN, _NDIM), lambda i: (0, i, 0)),
                  pl.BlockSpec((_NC, _BN, _NDIM), lambda i: (0, i, 0)),
                  full((_NDIM, _NDIM)), full((1, _NDIM)),
                  full((1, _NDIM)), full((1, _NDIM)),
                  full((_NDIM, 4 * _NDIM)), full((1, 4 * _NDIM)),
                  full((4 * _NDIM, _NDIM)), full((1, _NDIM)),
                  full((1, _NDIM)), full((1, _NDIM))],
        out_specs=pl.BlockSpec((_BN, _NDIM), lambda i: (i, 0)),
        out_shape=jax.ShapeDtypeStruct((_N, _NDIM), _f32),
    )(x, o_part, s_part, wo, bo, g1, b1, w1, f1, w2, f2, g2, b2)


def kernel(x, params, edge_feat, src_ids, dst_ids):
    p = params
    pad = _EPAD - _E
    src_p = jnp.pad(src_ids.astype(jnp.int32), (0, pad))
    dst_p = jnp.pad(dst_ids.astype(jnp.int32), (0, pad))
    feat_p = jnp.pad(edge_feat.astype(jnp.int32), (0, pad)).reshape(_EPAD, 1)

    gather = _make_gather()
    scatter = _make_scatter()
    o_zero = jnp.zeros((_NPAD, _NDIM), _f32)

    row = lambda v: v.reshape(1, -1)
    z16 = jnp.zeros((_NDIM, _EDIM), _f32)
    zpad = jnp.zeros((_NDIM, _SW - 2 * _NDIM - _EDIM), _f32)
    zpad_d = jnp.zeros((_NDIM, _DW - _NDIM - _EDIM), _f32)

    lg = jnp.zeros((_EPAD, _EDIM), _f32)
    for l in range(_L):
        w_src = p['aff_W'][l][_EDIM:_EDIM + _NDIM]
        w_dst = p['aff_W'][l][_EDIM + _NDIM:]
        wlg = p['aff_W'][l][:_EDIM]
        ws = jnp.concatenate([p['Wk'][l], p['Wv'][l], w_src, zpad], axis=1)
        bs = jnp.concatenate([p['bk'][l], p['bv'][l],
                              jnp.zeros((_SW - 2 * _NDIM,), _f32)])
        wd = jnp.concatenate([p['Wq'][l], w_dst, zpad_d], axis=1)
        bd = jnp.concatenate([p['bq'][l], jnp.zeros((_DW - _NDIM,), _f32)])
        st, dt = _proj_call(x, ws, row(bs), wd, row(bd))
        srcg, dstg = gather(st, dt, src_p, dst_p)
        use_feat = (l == 0)
        w, e128, lgn = _edge_call(
            use_feat, srcg, dstg, feat_p if use_feat else lg, p['rel_embed'],
            wlg, row(p['aff_b'][l]),
            row(p['ln_e1_g'][l]), row(p['ln_e1_b'][l]),
            p['ffn_e_W1'][l], row(p['ffn_e_b1'][l]),
            p['ffn_e_W2'][l], row(p['ffn_e_b2'][l]),
            row(p['ln_e2_g'][l]), row(p['ln_e2_b'][l]))
        o_part = scatter(w, dst_p, o_zero)
        s_part = scatter(e128, dst_p, o_zero)
        x = _node_call(
            x, o_part, s_part, p['Wo'][l], row(p['bo'][l]),
            row(p['ln_n1_g'][l]), row(p['ln_n1_b'][l]),
            p['ffn_n_W1'][l], row(p['ffn_n_b1'][l]),
            p['ffn_n_W2'][l], row(p['ffn_n_b2'][l]),
            row(p['ln_n2_g'][l]), row(p['ln_n2_b'][l]))
        lg = lgn
    return (x, lg[:_E])


# speedup vs baseline: 18.8974x; 1.1319x over previous
import functools

import jax
import jax.numpy as jnp
from jax import lax
from jax.experimental import pallas as pl
from jax.experimental.pallas import tpu as pltpu
from jax.experimental.pallas import tpu_sc as plsc

_N = 10000
_E = 160000
_NDIM = 128
_H = 8
_DH = 16
_EDIM = 16
_L = 2
_R = 128

_NC = 2
_NS = 16
_NW = _NC * _NS

_CH = 64
_NCHUNK = 80
_EW = _CH * _NCHUNK
_EPAD = _EW * _NW

_SW = 384
_DW = 256

_NPAD = 10240
_CHS = 128
_NCHUNK_S = _EW // _CHS

_f32 = jnp.float32


def _make_gather():
    widths = [_SW, _DW]
    nt = 2
    nb = 2
    mesh = plsc.VectorSubcoreMesh(core_axis_name="c", subcore_axis_name="s")
    out_type = [jax.ShapeDtypeStruct((_EPAD, w), _f32) for w in widths]
    scratch = (
        [pltpu.VMEM((_CH,), jnp.int32) for _ in widths for _ in range(nb)]
        + [pltpu.VMEM((_CH, w), _f32) for w in widths for _ in range(nb)]
        + [pltpu.SemaphoreType.DMA for _ in widths for _ in range(nb)]
    )

    @functools.partial(pl.kernel, out_type=out_type, mesh=mesh,
                       scratch_types=scratch)
    def gather_kernel(*refs):
        tables = refs[0:nt]
        idxs = refs[nt:2 * nt]
        outs = refs[2 * nt:3 * nt]
        k = 3 * nt
        ibufs = [refs[k + t * nb:k + (t + 1) * nb] for t in range(nt)]
        k += nt * nb
        rbufs = [refs[k + t * nb:k + (t + 1) * nb] for t in range(nt)]
        k += nt * nb
        sems = [refs[k + t * nb:k + (t + 1) * nb] for t in range(nt)]
        wid = lax.axis_index("s") * _NC + lax.axis_index("c")
        base = pl.multiple_of(wid * _EW, 8)

        def fire(i, b):
            off = pl.multiple_of(base + i * _CH, 8)
            for t in range(nt):
                pltpu.sync_copy(idxs[t].at[pl.ds(off, _CH)], ibufs[t][b])
                pltpu.async_copy(tables[t].at[ibufs[t][b]], rbufs[t][b],
                                 sems[t][b])

        def drain(i, b):
            off = pl.multiple_of(base + i * _CH, 8)
            for t in range(nt):
                pltpu.make_async_copy(tables[t].at[ibufs[t][b]], rbufs[t][b],
                                      sems[t][b]).wait()
                pltpu.sync_copy(rbufs[t][b], outs[t].at[pl.ds(off, _CH)])

        for i in range(nb):
            fire(i, i)

        def body(j, carry):
            i0 = j * nb
            for b in range(nb):
                i = i0 + b
                drain(i, b)

                @pl.when(i + nb < _NCHUNK)
                def _next():
                    fire(i + nb, b)
            return carry

        lax.fori_loop(0, _NCHUNK // nb, body, 0)

    return gather_kernel


def _make_scatter():
    mesh = plsc.VectorSubcoreMesh(core_axis_name="c", subcore_axis_name="s")
    out_type = jax.ShapeDtypeStruct((_NC, _NPAD, _NDIM), _f32)
    scratch = [
        pltpu.VMEM_SHARED((_NPAD, _NDIM), _f32),
        pltpu.VMEM((_CHS,), jnp.int32),
        pltpu.VMEM((_CHS, _NDIM), _f32),
    ]

    @functools.partial(pl.kernel, out_type=out_type, mesh=mesh,
                       scratch_types=scratch)
    def scatter_kernel(w_hbm, dst_hbm, oz_hbm, o_out, o_sh, ib, wb):
        c = lax.axis_index("c")
        s = lax.axis_index("s")

        @pl.when(s == 0)
        def _init():
            pltpu.sync_copy(oz_hbm, o_sh)

        plsc.subcore_barrier()

        base = pl.multiple_of((c * _NS + s) * _EW, 8)

        def body(i, carry):
            off = pl.multiple_of(base + i * _CHS, 8)
            pltpu.sync_copy(dst_hbm.at[pl.ds(off, _CHS)], ib)
            pltpu.sync_copy(w_hbm.at[pl.ds(off, _CHS)], wb)
            pltpu.sync_copy(wb, o_sh.at[ib], add=True)
            return carry

        lax.fori_loop(0, _NCHUNK_S, body, 0)
        plsc.subcore_barrier()

        @pl.when(s == 0)
        def _writeout():
            pltpu.sync_copy(o_sh, o_out.at[c])

    return scatter_kernel


_BN = 1000
_BE = 2048


def _proj_body(x_ref, ws, bs, wd, bd, st_ref, dt_ref):
    x = x_ref[...]
    st_ref[...] = jnp.dot(x, ws[...], preferred_element_type=_f32) + bs[...]
    dt_ref[...] = jnp.dot(x, wd[...], preferred_element_type=_f32) + bd[...]


def _proj_call(x, ws, bs, wd, bd):
    grid = (_N // _BN,)
    full = lambda shape: pl.BlockSpec(shape, lambda i: (0,) * len(shape))
    rows = lambda w: pl.BlockSpec((_BN, w), lambda i: (i, 0))
    return pl.pallas_call(
        _proj_body,
        grid=grid,
        in_specs=[rows(_NDIM), full((_NDIM, _SW)), full((1, _SW)),
                  full((_NDIM, _DW)), full((1, _DW))],
        out_specs=[rows(_SW), rows(_DW)],
        out_shape=[jax.ShapeDtypeStruct((_N, _SW), _f32),
                   jax.ShapeDtypeStruct((_N, _DW), _f32)],
    )(x, ws, bs, wd, bd)


def _ln_mm(x, ones_mm, g, b):
    m = jnp.dot(x, ones_mm, preferred_element_type=_f32)
    v = jnp.dot(x * x, ones_mm, preferred_element_type=_f32) - m * m
    return (x - m) / jnp.sqrt(v + 1e-5) * g + b


def _edge_body(use_feat, ks, vs, xss, qd, xdd, lg_in, rel,
               wlg, ba, ge1, be1, w1, b1, w2, b2, ge2, be2,
               w_ref, e128_ref, lgn_ref):
    r16 = lax.broadcasted_iota(jnp.int32, (_EDIM, _NDIM), 0)
    c16 = lax.broadcasted_iota(jnp.int32, (_EDIM, _NDIM), 1)
    t16 = (c16 % _EDIM == r16).astype(_f32)
    rs = lax.broadcasted_iota(jnp.int32, (_NDIM, _H), 0)
    cs = lax.broadcasted_iota(jnp.int32, (_NDIM, _H), 1)
    sel = (rs // _DH == cs).astype(_f32)
    rb = lax.broadcasted_iota(jnp.int32, (_H, _NDIM), 0)
    cb = lax.broadcasted_iota(jnp.int32, (_H, _NDIM), 1)
    selt = (cb // _DH == rb).astype(_f32)
    re = lax.broadcasted_iota(jnp.int32, (_NDIM, _EDIM), 0)
    ce = lax.broadcasted_iota(jnp.int32, (_NDIM, _EDIM), 1)
    ext16 = (re == ce).astype(_f32)
    xss = jnp.dot(xss[...], ext16, preferred_element_type=_f32)
    xdd = jnp.dot(xdd[...], ext16, preferred_element_type=_f32)

    if use_feat:
        lanes = lax.broadcasted_iota(jnp.int32, (_BE, _R), 1)
        onehot = (lg_in[...] == lanes).astype(_f32)
        lgb = jnp.dot(onehot, rel[...], preferred_element_type=_f32)
    else:
        lgb = lg_in[...]
    lg128 = jnp.dot(lgb, t16, preferred_element_type=_f32)
    t = qd[...] * (ks[...] + lg128)
    esc = jnp.exp(jnp.dot(t, sel, preferred_element_type=_f32) * 0.25)
    i = pl.program_id(0)
    rid = i * _BE + lax.broadcasted_iota(jnp.int32, (_BE, _H), 0)
    esc = jnp.where(rid < _E, esc, 0.0)
    e128 = jnp.dot(esc, selt, preferred_element_type=_f32)
    w_ref[...] = (vs[...] + lg128) * e128
    e128_ref[...] = e128

    one16 = jnp.full((_EDIM, _EDIM), 1.0 / _EDIM, _f32)
    a = jnp.tanh(jnp.dot(lgb, wlg[...], preferred_element_type=_f32)
                 + xss + xdd + ba[...])
    eh = _ln_mm(lgb + a, one16, ge1[...], be1[...])
    ff = jnp.dot(jnp.maximum(jnp.dot(eh, w1[...], preferred_element_type=_f32)
                             + b1[...], 0.0),
                 w2[...], preferred_element_type=_f32) + b2[...]
    lgn_ref[...] = _ln_mm(eh + ff, one16, ge2[...], be2[...])


def _edge_call(use_feat, srcg, dstg, lg_or_feat, rel,
               wlg, ba, ge1, be1, w1, b1, w2, b2, ge2, be2):
    grid = (_EPAD // _BE,)
    full = lambda shape: pl.BlockSpec(shape, lambda i: (0,) * len(shape))
    rows = lambda w: pl.BlockSpec((_BE, w), lambda i: (i, 0))
    lg_spec = (pl.BlockSpec((_BE, 1), lambda i: (i, 0)) if use_feat
               else rows(_EDIM))
    return pl.pallas_call(
        functools.partial(_edge_body, use_feat),
        grid=grid,
        in_specs=[pl.BlockSpec((_BE, _NDIM), lambda i: (i, 0)),
                  pl.BlockSpec((_BE, _NDIM), lambda i: (i, 1)),
                  pl.BlockSpec((_BE, _NDIM), lambda i: (i, 2)),
                  pl.BlockSpec((_BE, _NDIM), lambda i: (i, 0)),
                  pl.BlockSpec((_BE, _NDIM), lambda i: (i, 1)),
                  lg_spec,
                  full((_R, _EDIM)),
                  full((_EDIM, _EDIM)), full((1, _EDIM)),
                  full((1, _EDIM)), full((1, _EDIM)),
                  full((_EDIM, 4 * _EDIM)), full((1, 4 * _EDIM)),
                  full((4 * _EDIM, _EDIM)), full((1, _EDIM)),
                  full((1, _EDIM)), full((1, _EDIM))],
        out_specs=[rows(_NDIM), rows(_NDIM), rows(_EDIM)],
        out_shape=[jax.ShapeDtypeStruct((_EPAD, _NDIM), _f32),
                   jax.ShapeDtypeStruct((_EPAD, _NDIM), _f32),
                   jax.ShapeDtypeStruct((_EPAD, _EDIM), _f32)],
    )(srcg, srcg, srcg, dstg, dstg, lg_or_feat, rel,
      wlg, ba, ge1, be1, w1, b1, w2, b2, ge2, be2)


def _node_body(x_ref, op, sp, wo, bo, g1, b1, w1, f1, w2, f2, g2, b2,
               out_ref):
    o_un = op[0] + op[1]
    sv = sp[0] + sp[1]
    o = o_un / (sv + 1e-12)
    o = jnp.dot(o, wo[...], preferred_element_type=_f32) + bo[...]
    one128 = jnp.full((_NDIM, _NDIM), 1.0 / _NDIM, _f32)
    h = _ln_mm(x_ref[...] + o, one128, g1[...], b1[...])
    ff = jnp.dot(jnp.maximum(jnp.dot(h, w1[...], preferred_element_type=_f32)
                             + f1[...], 0.0),
                 w2[...], preferred_element_type=_f32) + f2[...]
    out_ref[...] = _ln_mm(h + ff, one128, g2[...], b2[...])


def _node_call(x, o_part, s_part, wo, bo, g1, b1, w1, f1, w2, f2, g2, b2):
    grid = (_N // _BN,)
    full = lambda shape: pl.BlockSpec(shape, lambda i: (0,) * len(shape))
    return pl.pallas_call(
        _node_body,
        grid=grid,
        in_specs=[pl.BlockSpec((_BN, _NDIM), lambda i: (i, 0)),
                  pl.BlockSpec((_NC, _BN, _NDIM), lambda i: (0, i, 0)),
                  pl.BlockSpec((_NC, _BN, _NDIM), lambda i: (0, i, 0)),
                  full((_NDIM, _NDIM)), full((1, _NDIM)),
                  full((1, _NDIM)), full((1, _NDIM)),
                  full((_NDIM, 4 * _NDIM)), full((1, 4 * _NDIM)),
                  full((4 * _NDIM, _NDIM)), full((1, _NDIM)),
                  full((1, _NDIM)), full((1, _NDIM))],
        out_specs=pl.BlockSpec((_BN, _NDIM), lambda i: (i, 0)),
        out_shape=jax.ShapeDtypeStruct((_N, _NDIM), _f32),
    )(x, o_part, s_part, wo, bo, g1, b1, w1, f1, w2, f2, g2, b2)


def kernel(x, params, edge_feat, src_ids, dst_ids):
    p = params
    pad = _EPAD - _E
    src_p = jnp.pad(src_ids.astype(jnp.int32), (0, pad))
    dst_p = jnp.pad(dst_ids.astype(jnp.int32), (0, pad))
    feat_p = jnp.pad(edge_feat.astype(jnp.int32), (0, pad)).reshape(_EPAD, 1)

    gather = _make_gather()
    scatter = _make_scatter()
    o_zero = jnp.zeros((_NPAD, _NDIM), _f32)

    row = lambda v: v.reshape(1, -1)
    z16 = jnp.zeros((_NDIM, _EDIM), _f32)
    zpad = jnp.zeros((_NDIM, _SW - 2 * _NDIM - _EDIM), _f32)
    zpad_d = jnp.zeros((_NDIM, _DW - _NDIM - _EDIM), _f32)

    lg = jnp.zeros((_EPAD, _EDIM), _f32)
    for l in range(_L):
        w_src = p['aff_W'][l][_EDIM:_EDIM + _NDIM]
        w_dst = p['aff_W'][l][_EDIM + _NDIM:]
        wlg = p['aff_W'][l][:_EDIM]
        ws = jnp.concatenate([p['Wk'][l], p['Wv'][l], w_src, zpad], axis=1)
        bs = jnp.concatenate([p['bk'][l], p['bv'][l],
                              jnp.zeros((_SW - 2 * _NDIM,), _f32)])
        wd = jnp.concatenate([p['Wq'][l], w_dst, zpad_d], axis=1)
        bd = jnp.concatenate([p['bq'][l], jnp.zeros((_DW - _NDIM,), _f32)])
        st, dt = _proj_call(x, ws, row(bs), wd, row(bd))
        srcg, dstg = gather(st, dt, src_p, dst_p)
        use_feat = (l == 0)
        w, e128, lgn = _edge_call(
            use_feat, srcg, dstg, feat_p if use_feat else lg, p['rel_embed'],
            wlg, row(p['aff_b'][l]),
            row(p['ln_e1_g'][l]), row(p['ln_e1_b'][l]),
            p['ffn_e_W1'][l], row(p['ffn_e_b1'][l]),
            p['ffn_e_W2'][l], row(p['ffn_e_b2'][l]),
            row(p['ln_e2_g'][l]), row(p['ln_e2_b'][l]))
        o_part = scatter(w, dst_p, o_zero)
        s_part = scatter(e128, dst_p, o_zero)
        x = _node_call(
            x, o_part, s_part, p['Wo'][l], row(p['bo'][l]),
            row(p['ln_n1_g'][l]), row(p['ln_n1_b'][l]),
            p['ffn_n_W1'][l], row(p['ffn_n_b1'][l]),
            p['ffn_n_W2'][l], row(p['ffn_n_b2'][l]),
            row(p['ln_n2_g'][l]), row(p['ln_n2_b'][l]))
        lg = lgn
    return (x, lg[:_E])


# speedup vs baseline: 20.7068x; 1.0957x over previous
import functools

import jax
import jax.numpy as jnp
from jax import lax
from jax.experimental import pallas as pl
from jax.experimental.pallas import tpu as pltpu
from jax.experimental.pallas import tpu_sc as plsc

_N = 10000
_E = 160000
_NDIM = 128
_H = 8
_DH = 16
_EDIM = 16
_L = 2
_R = 128

_NC = 2
_NS = 16
_NW = _NC * _NS

_CH = 64
_NCHUNK = 80
_EW = _CH * _NCHUNK
_EPAD = _EW * _NW

_SW = 384
_DW = 256

_NPAD = 10240
_CHS = 128
_NCHUNK_S = _EW // _CHS

_f32 = jnp.float32


def _make_gather():
    widths = [_SW, _DW]
    nt = 2
    nb = 2
    mesh = plsc.VectorSubcoreMesh(core_axis_name="c", subcore_axis_name="s")
    out_type = [jax.ShapeDtypeStruct((_EPAD, w), _f32) for w in widths]
    scratch = (
        [pltpu.VMEM((_NCHUNK, _CH), jnp.int32) for _ in widths]
        + [pltpu.VMEM((_CH, w), _f32) for w in widths for _ in range(nb)]
        + [pltpu.SemaphoreType.DMA for _ in widths for _ in range(nb)]
    )

    @functools.partial(pl.kernel, out_type=out_type, mesh=mesh,
                       scratch_types=scratch)
    def gather_kernel(*refs):
        tables = refs[0:nt]
        idxs = refs[nt:2 * nt]
        outs = refs[2 * nt:3 * nt]
        k = 3 * nt
        islabs = refs[k:k + nt]
        k += nt
        rbufs = [refs[k + t * nb:k + (t + 1) * nb] for t in range(nt)]
        k += nt * nb
        sems = [refs[k + t * nb:k + (t + 1) * nb] for t in range(nt)]
        wid = lax.axis_index("s") * _NC + lax.axis_index("c")
        base = pl.multiple_of(wid * _EW, 8)
        crow = pl.multiple_of(wid * _NCHUNK, 8)

        for t in range(nt):
            pltpu.sync_copy(idxs[t].at[pl.ds(crow, _NCHUNK)], islabs[t])

        def fire(i, b):
            for t in range(nt):
                pltpu.async_copy(tables[t].at[islabs[t].at[i]], rbufs[t][b],
                                 sems[t][b])

        def drain(i, b):
            off = pl.multiple_of(base + i * _CH, 8)
            for t in range(nt):
                pltpu.make_async_copy(tables[t].at[islabs[t].at[i]],
                                      rbufs[t][b], sems[t][b]).wait()
                pltpu.sync_copy(rbufs[t][b], outs[t].at[pl.ds(off, _CH)])

        for i in range(nb):
            fire(i, i)

        def body(j, carry):
            i0 = j * nb
            for b in range(nb):
                i = i0 + b
                drain(i, b)

                @pl.when(i + nb < _NCHUNK)
                def _next():
                    fire(i + nb, b)
            return carry

        lax.fori_loop(0, _NCHUNK // nb, body, 0)

    return gather_kernel


_EW2 = _EPAD // _NS
_NCH2 = _EW2 // _CHS


def _make_scatter():
    nb = 2
    mesh = plsc.VectorSubcoreMesh(core_axis_name="c", subcore_axis_name="s")
    out_type = [
        jax.ShapeDtypeStruct((_NPAD, _NDIM), _f32),
        jax.ShapeDtypeStruct((_NPAD, _NDIM), _f32),
    ]
    scratch = (
        [pltpu.VMEM_SHARED((_NPAD, _NDIM), _f32),
         pltpu.VMEM((_NCH2, _CHS), jnp.int32)]
        + [pltpu.VMEM((_CHS, _NDIM), _f32) for _ in range(nb)]
        + [pltpu.SemaphoreType.DMA for _ in range(2 * nb)]
    )

    @functools.partial(pl.kernel, out_type=out_type, mesh=mesh,
                       scratch_types=scratch)
    def scatter_kernel(w_hbm, e_hbm, dst_hbm, oz_hbm, o_out, s_out,
                       acc, islab, wb0, wb1, ls0, ls1, as0, as1):
        c = lax.axis_index("c")
        s = lax.axis_index("s")
        wbs = [wb0, wb1]
        lsems = [ls0, ls1]
        asems = [as0, as1]

        @pl.when(s == 0)
        def _init():
            pltpu.sync_copy(oz_hbm, acc)

        crow = pl.multiple_of(s * _NCH2, 8)
        pltpu.sync_copy(dst_hbm.at[pl.ds(crow, _NCH2)], islab)
        plsc.subcore_barrier()

        base = pl.multiple_of(s * _EW2, 8)

        def run(arr_hbm):
            def fire_load(i, b):
                off = pl.multiple_of(base + i * _CHS, 8)
                pltpu.async_copy(arr_hbm.at[pl.ds(off, _CHS)], wbs[b],
                                 lsems[b])

            def step(i, b):
                off = pl.multiple_of(base + i * _CHS, 8)
                pltpu.make_async_copy(arr_hbm.at[pl.ds(off, _CHS)], wbs[b],
                                      lsems[b]).wait()
                pltpu.async_copy(wbs[b], acc.at[islab.at[i]], asems[b],
                                 add=True)

                @pl.when(i + nb < _NCH2)
                def _next():
                    pltpu.make_async_copy(wbs[b], acc.at[islab.at[i]],
                                          asems[b]).wait()
                    fire_load(i + nb, b)

            for i in range(nb):
                fire_load(i, i)

            def body(j, carry):
                for b in range(nb):
                    step(j * nb + b, b)
                return carry

            lax.fori_loop(0, _NCH2 // nb, body, 0)
            for b in range(nb):
                pltpu.make_async_copy(wbs[b], acc.at[islab.at[_NCH2 - nb + b]],
                                      asems[b]).wait()

        @pl.when(c == 0)
        def _scatter_w():
            run(w_hbm)

        @pl.when(c == 1)
        def _scatter_e():
            run(e_hbm)

        plsc.subcore_barrier()

        @pl.when((s == 0) & (c == 0))
        def _write_o():
            pltpu.sync_copy(acc, o_out)

        @pl.when((s == 0) & (c == 1))
        def _write_s():
            pltpu.sync_copy(acc, s_out)

    return scatter_kernel


_BN = 1000
_BE = 2048


def _proj_body(x_ref, ws, bs, wd, bd, st_ref, dt_ref):
    x = x_ref[...]
    st_ref[...] = jnp.dot(x, ws[...], preferred_element_type=_f32) + bs[...]
    dt_ref[...] = jnp.dot(x, wd[...], preferred_element_type=_f32) + bd[...]


def _proj_call(x, ws, bs, wd, bd):
    grid = (_N // _BN,)
    full = lambda shape: pl.BlockSpec(shape, lambda i: (0,) * len(shape))
    rows = lambda w: pl.BlockSpec((_BN, w), lambda i: (i, 0))
    return pl.pallas_call(
        _proj_body,
        grid=grid,
        in_specs=[rows(_NDIM), full((_NDIM, _SW)), full((1, _SW)),
                  full((_NDIM, _DW)), full((1, _DW))],
        out_specs=[rows(_SW), rows(_DW)],
        out_shape=[jax.ShapeDtypeStruct((_N, _SW), _f32),
                   jax.ShapeDtypeStruct((_N, _DW), _f32)],
    )(x, ws, bs, wd, bd)


def _ln_mm(x, ones_mm, g, b):
    m = jnp.dot(x, ones_mm, preferred_element_type=_f32)
    v = jnp.dot(x * x, ones_mm, preferred_element_type=_f32) - m * m
    return (x - m) / jnp.sqrt(v + 1e-5) * g + b


def _edge_body(use_feat, ks, vs, xss, qd, xdd, lg_in, rel,
               wlg, ba, ge1, be1, w1, b1, w2, b2, ge2, be2,
               w_ref, e128_ref, lgn_ref):
    r16 = lax.broadcasted_iota(jnp.int32, (_EDIM, _NDIM), 0)
    c16 = lax.broadcasted_iota(jnp.int32, (_EDIM, _NDIM), 1)
    t16 = (c16 % _EDIM == r16).astype(_f32)
    rs = lax.broadcasted_iota(jnp.int32, (_NDIM, _H), 0)
    cs = lax.broadcasted_iota(jnp.int32, (_NDIM, _H), 1)
    sel = (rs // _DH == cs).astype(_f32)
    rb = lax.broadcasted_iota(jnp.int32, (_H, _NDIM), 0)
    cb = lax.broadcasted_iota(jnp.int32, (_H, _NDIM), 1)
    selt = (cb // _DH == rb).astype(_f32)
    re = lax.broadcasted_iota(jnp.int32, (_NDIM, _EDIM), 0)
    ce = lax.broadcasted_iota(jnp.int32, (_NDIM, _EDIM), 1)
    ext16 = (re == ce).astype(_f32)
    xss = jnp.dot(xss[...], ext16, preferred_element_type=_f32)
    xdd = jnp.dot(xdd[...], ext16, preferred_element_type=_f32)

    if use_feat:
        lanes = lax.broadcasted_iota(jnp.int32, (_BE, _R), 1)
        onehot = (lg_in[...] == lanes).astype(_f32)
        lgb = jnp.dot(onehot, rel[...], preferred_element_type=_f32)
    else:
        lgb = lg_in[...]
    lg128 = jnp.dot(lgb, t16, preferred_element_type=_f32)
    t = qd[...] * (ks[...] + lg128)
    esc = jnp.exp(jnp.dot(t, sel, preferred_element_type=_f32) * 0.25)
    i = pl.program_id(0)
    rid = i * _BE + lax.broadcasted_iota(jnp.int32, (_BE, _H), 0)
    esc = jnp.where(rid < _E, esc, 0.0)
    e128 = jnp.dot(esc, selt, preferred_element_type=_f32)
    w_ref[...] = (vs[...] + lg128) * e128
    e128_ref[...] = e128

    one16 = jnp.full((_EDIM, _EDIM), 1.0 / _EDIM, _f32)
    a = jnp.tanh(jnp.dot(lgb, wlg[...], preferred_element_type=_f32)
                 + xss + xdd + ba[...])
    eh = _ln_mm(lgb + a, one16, ge1[...], be1[...])
    ff = jnp.dot(jnp.maximum(jnp.dot(eh, w1[...], preferred_element_type=_f32)
                             + b1[...], 0.0),
                 w2[...], preferred_element_type=_f32) + b2[...]
    lgn_ref[...] = _ln_mm(eh + ff, one16, ge2[...], be2[...])


def _edge_call(use_feat, srcg, dstg, lg_or_feat, rel,
               wlg, ba, ge1, be1, w1, b1, w2, b2, ge2, be2):
    grid = (_EPAD // _BE,)
    full = lambda shape: pl.BlockSpec(shape, lambda i: (0,) * len(shape))
    rows = lambda w: pl.BlockSpec((_BE, w), lambda i: (i, 0))
    lg_spec = (pl.BlockSpec((_BE, 1), lambda i: (i, 0)) if use_feat
               else rows(_EDIM))
    return pl.pallas_call(
        functools.partial(_edge_body, use_feat),
        grid=grid,
        in_specs=[pl.BlockSpec((_BE, _NDIM), lambda i: (i, 0)),
                  pl.BlockSpec((_BE, _NDIM), lambda i: (i, 1)),
                  pl.BlockSpec((_BE, _NDIM), lambda i: (i, 2)),
                  pl.BlockSpec((_BE, _NDIM), lambda i: (i, 0)),
                  pl.BlockSpec((_BE, _NDIM), lambda i: (i, 1)),
                  lg_spec,
                  full((_R, _EDIM)),
                  full((_EDIM, _EDIM)), full((1, _EDIM)),
                  full((1, _EDIM)), full((1, _EDIM)),
                  full((_EDIM, 4 * _EDIM)), full((1, 4 * _EDIM)),
                  full((4 * _EDIM, _EDIM)), full((1, _EDIM)),
                  full((1, _EDIM)), full((1, _EDIM))],
        out_specs=[rows(_NDIM), rows(_NDIM), rows(_EDIM)],
        out_shape=[jax.ShapeDtypeStruct((_EPAD, _NDIM), _f32),
                   jax.ShapeDtypeStruct((_EPAD, _NDIM), _f32),
                   jax.ShapeDtypeStruct((_EPAD, _EDIM), _f32)],
    )(srcg, srcg, srcg, dstg, dstg, lg_or_feat, rel,
      wlg, ba, ge1, be1, w1, b1, w2, b2, ge2, be2)


def _node_body(x_ref, op, sp, wo, bo, g1, b1, w1, f1, w2, f2, g2, b2,
               out_ref):
    o_un = op[...]
    sv = sp[...]
    o = o_un / (sv + 1e-12)
    o = jnp.dot(o, wo[...], preferred_element_type=_f32) + bo[...]
    one128 = jnp.full((_NDIM, _NDIM), 1.0 / _NDIM, _f32)
    h = _ln_mm(x_ref[...] + o, one128, g1[...], b1[...])
    ff = jnp.dot(jnp.maximum(jnp.dot(h, w1[...], preferred_element_type=_f32)
                             + f1[...], 0.0),
                 w2[...], preferred_element_type=_f32) + f2[...]
    out_ref[...] = _ln_mm(h + ff, one128, g2[...], b2[...])


def _node_call(x, o_part, s_part, wo, bo, g1, b1, w1, f1, w2, f2, g2, b2):
    grid = (_N // _BN,)
    full = lambda shape: pl.BlockSpec(shape, lambda i: (0,) * len(shape))
    return pl.pallas_call(
        _node_body,
        grid=grid,
        in_specs=[pl.BlockSpec((_BN, _NDIM), lambda i: (i, 0)),
                  pl.BlockSpec((_BN, _NDIM), lambda i: (i, 0)),
                  pl.BlockSpec((_BN, _NDIM), lambda i: (i, 0)),
                  full((_NDIM, _NDIM)), full((1, _NDIM)),
                  full((1, _NDIM)), full((1, _NDIM)),
                  full((_NDIM, 4 * _NDIM)), full((1, 4 * _NDIM)),
                  full((4 * _NDIM, _NDIM)), full((1, _NDIM)),
                  full((1, _NDIM)), full((1, _NDIM))],
        out_specs=pl.BlockSpec((_BN, _NDIM), lambda i: (i, 0)),
        out_shape=jax.ShapeDtypeStruct((_N, _NDIM), _f32),
    )(x, o_part, s_part, wo, bo, g1, b1, w1, f1, w2, f2, g2, b2)


def kernel(x, params, edge_feat, src_ids, dst_ids):
    p = params
    pad = _EPAD - _E
    src_p = jnp.pad(src_ids.astype(jnp.int32), (0, pad))
    dst_p = jnp.pad(dst_ids.astype(jnp.int32), (0, pad))
    feat_p = jnp.pad(edge_feat.astype(jnp.int32), (0, pad)).reshape(_EPAD, 1)
    src_g = src_p.reshape(_EPAD // _CH, _CH)
    dst_g = dst_p.reshape(_EPAD // _CH, _CH)
    dst_s = dst_p.reshape(_EPAD // _CHS, _CHS)

    gather = _make_gather()
    scatter = _make_scatter()
    o_zero = jnp.zeros((_NPAD, _NDIM), _f32)

    row = lambda v: v.reshape(1, -1)
    z16 = jnp.zeros((_NDIM, _EDIM), _f32)
    zpad = jnp.zeros((_NDIM, _SW - 2 * _NDIM - _EDIM), _f32)
    zpad_d = jnp.zeros((_NDIM, _DW - _NDIM - _EDIM), _f32)

    lg = jnp.zeros((_EPAD, _EDIM), _f32)
    for l in range(_L):
        w_src = p['aff_W'][l][_EDIM:_EDIM + _NDIM]
        w_dst = p['aff_W'][l][_EDIM + _NDIM:]
        wlg = p['aff_W'][l][:_EDIM]
        ws = jnp.concatenate([p['Wk'][l], p['Wv'][l], w_src, zpad], axis=1)
        bs = jnp.concatenate([p['bk'][l], p['bv'][l],
                              jnp.zeros((_SW - 2 * _NDIM,), _f32)])
        wd = jnp.concatenate([p['Wq'][l], w_dst, zpad_d], axis=1)
        bd = jnp.concatenate([p['bq'][l], jnp.zeros((_DW - _NDIM,), _f32)])
        st, dt = _proj_call(x, ws, row(bs), wd, row(bd))
        srcg, dstg = gather(st, dt, src_g, dst_g)
        use_feat = (l == 0)
        w, e128, lgn = _edge_call(
            use_feat, srcg, dstg, feat_p if use_feat else lg, p['rel_embed'],
            wlg, row(p['aff_b'][l]),
            row(p['ln_e1_g'][l]), row(p['ln_e1_b'][l]),
            p['ffn_e_W1'][l], row(p['ffn_e_b1'][l]),
            p['ffn_e_W2'][l], row(p['ffn_e_b2'][l]),
            row(p['ln_e2_g'][l]), row(p['ln_e2_b'][l]))
        o_part, s_part = scatter(w, e128, dst_s, o_zero)
        x = _node_call(
            x, o_part, s_part, p['Wo'][l], row(p['bo'][l]),
            row(p['ln_n1_g'][l]), row(p['ln_n1_b'][l]),
            p['ffn_n_W1'][l], row(p['ffn_n_b1'][l]),
            p['ffn_n_W2'][l], row(p['ffn_n_b2'][l]),
            row(p['ln_n2_g'][l]), row(p['ln_n2_b'][l]))
        lg = lgn
    return (x, lg[:_E])


# speedup vs baseline: 24.6089x; 1.1884x over previous
import functools

import jax
import jax.numpy as jnp
from jax import lax
from jax.experimental import pallas as pl
from jax.experimental.pallas import tpu as pltpu
from jax.experimental.pallas import tpu_sc as plsc

_N = 10000
_E = 160000
_NDIM = 128
_H = 8
_DH = 16
_EDIM = 16
_L = 2
_R = 128

_NC = 2
_NS = 16
_NW = _NC * _NS

_CH = 128
_NCHUNK = 40
_EW = _CH * _NCHUNK
_EPAD = _EW * _NW

_SW = 384
_DW = 256
_SWP = 256
_DWP = 128

_NPAD = 10240
_CHS = 128
_NCHUNK_S = _EW // _CHS

_f32 = jnp.float32
_bf16 = jnp.bfloat16


def _make_gather():
    widths = [_SWP, _DWP]
    nt = 2
    nb = 2
    mesh = plsc.VectorSubcoreMesh(core_axis_name="c", subcore_axis_name="s")
    out_type = [jax.ShapeDtypeStruct((_EPAD, w), jnp.int32) for w in widths]
    scratch = (
        [pltpu.VMEM((_NCHUNK, _CH), jnp.int32) for _ in widths]
        + [pltpu.VMEM((_CH, w), jnp.int32) for w in widths for _ in range(nb)]
        + [pltpu.SemaphoreType.DMA for _ in widths for _ in range(nb)]
    )

    @functools.partial(pl.kernel, out_type=out_type, mesh=mesh,
                       scratch_types=scratch)
    def gather_kernel(*refs):
        tables = refs[0:nt]
        idxs = refs[nt:2 * nt]
        outs = refs[2 * nt:3 * nt]
        k = 3 * nt
        islabs = refs[k:k + nt]
        k += nt
        rbufs = [refs[k + t * nb:k + (t + 1) * nb] for t in range(nt)]
        k += nt * nb
        sems = [refs[k + t * nb:k + (t + 1) * nb] for t in range(nt)]
        wid = lax.axis_index("s") * _NC + lax.axis_index("c")
        base = pl.multiple_of(wid * _EW, 8)
        crow = pl.multiple_of(wid * _NCHUNK, 8)

        for t in range(nt):
            pltpu.sync_copy(idxs[t].at[pl.ds(crow, _NCHUNK)], islabs[t])

        def fire(i, b):
            for t in range(nt):
                pltpu.async_copy(tables[t].at[islabs[t].at[i]], rbufs[t][b],
                                 sems[t][b])

        def drain(i, b):
            off = pl.multiple_of(base + i * _CH, 8)
            for t in range(nt):
                pltpu.make_async_copy(tables[t].at[islabs[t].at[i]],
                                      rbufs[t][b], sems[t][b]).wait()
                pltpu.sync_copy(rbufs[t][b], outs[t].at[pl.ds(off, _CH)])

        for i in range(nb):
            fire(i, i)

        def body(j, carry):
            i0 = j * nb
            for b in range(nb):
                i = i0 + b
                drain(i, b)

                @pl.when(i + nb < _NCHUNK)
                def _next():
                    fire(i + nb, b)
            return carry

        lax.fori_loop(0, _NCHUNK // nb, body, 0)

    return gather_kernel


_EW2 = _EPAD // _NS
_NCH2 = _EW2 // _CHS


def _make_scatter():
    nb = 2
    mesh = plsc.VectorSubcoreMesh(core_axis_name="c", subcore_axis_name="s")
    out_type = [
        jax.ShapeDtypeStruct((_NPAD, _NDIM), _f32),
        jax.ShapeDtypeStruct((_NPAD, _NDIM), _f32),
    ]
    scratch = (
        [pltpu.VMEM_SHARED((_NPAD, _NDIM), _f32),
         pltpu.VMEM((_NCH2, _CHS), jnp.int32)]
        + [pltpu.VMEM((_CHS, _NDIM), _f32) for _ in range(nb)]
        + [pltpu.SemaphoreType.DMA for _ in range(2 * nb)]
    )

    @functools.partial(pl.kernel, out_type=out_type, mesh=mesh,
                       scratch_types=scratch)
    def scatter_kernel(w_hbm, e_hbm, dst_hbm, oz_hbm, o_out, s_out,
                       acc, islab, wb0, wb1, ls0, ls1, as0, as1):
        c = lax.axis_index("c")
        s = lax.axis_index("s")
        wbs = [wb0, wb1]
        lsems = [ls0, ls1]
        asems = [as0, as1]

        @pl.when(s == 0)
        def _init():
            pltpu.sync_copy(oz_hbm, acc)

        crow = pl.multiple_of(s * _NCH2, 8)
        pltpu.sync_copy(dst_hbm.at[pl.ds(crow, _NCH2)], islab)
        plsc.subcore_barrier()

        base = pl.multiple_of(s * _EW2, 8)

        def run(arr_hbm):
            def fire_load(i, b):
                off = pl.multiple_of(base + i * _CHS, 8)
                pltpu.async_copy(arr_hbm.at[pl.ds(off, _CHS)], wbs[b],
                                 lsems[b])

            def step(i, b):
                off = pl.multiple_of(base + i * _CHS, 8)
                pltpu.make_async_copy(arr_hbm.at[pl.ds(off, _CHS)], wbs[b],
                                      lsems[b]).wait()
                pltpu.async_copy(wbs[b], acc.at[islab.at[i]], asems[b],
                                 add=True)

                @pl.when(i + nb < _NCH2)
                def _next():
                    pltpu.make_async_copy(wbs[b], acc.at[islab.at[i]],
                                          asems[b]).wait()
                    fire_load(i + nb, b)

            for i in range(nb):
                fire_load(i, i)

            def body(j, carry):
                for b in range(nb):
                    step(j * nb + b, b)
                return carry

            lax.fori_loop(0, _NCH2 // nb, body, 0)
            for b in range(nb):
                pltpu.make_async_copy(wbs[b], acc.at[islab.at[_NCH2 - nb + b]],
                                      asems[b]).wait()

        @pl.when(c == 0)
        def _scatter_w():
            run(w_hbm)

        @pl.when(c == 1)
        def _scatter_e():
            run(e_hbm)

        plsc.subcore_barrier()

        @pl.when((s == 0) & (c == 0))
        def _write_o():
            pltpu.sync_copy(acc, o_out)

        @pl.when((s == 0) & (c == 1))
        def _write_s():
            pltpu.sync_copy(acc, s_out)

    return scatter_kernel


_BN = 1000
_BE = 2048


def _bits16(x):
    y = x.astype(_bf16).astype(_f32)
    return lax.bitcast_convert_type(y, jnp.int32)


def _proj_body(x_ref, ws, bs, wd, bd, st_ref, dt_ref):
    x = x_ref[...]
    accs = jnp.dot(x, ws[...], preferred_element_type=_f32) + bs[...]
    accd = jnp.dot(x, wd[...], preferred_element_type=_f32) + bd[...]
    kp, vp, xsb = accs[:, :128], accs[:, 128:256], accs[:, 256:384]
    qp, xdb = accd[:, :128], accd[:, 128:256]
    hi_mask = jnp.int32(-65536)
    pk1 = (_bits16(vp) & hi_mask) | ((_bits16(kp) >> 16) & 0xFFFF)
    pk2 = (_bits16(xsb) >> 16) & 0xFFFF
    st_ref[...] = jnp.concatenate([pk1, pk2], axis=1)
    dt_ref[...] = (_bits16(xdb) & hi_mask) | ((_bits16(qp) >> 16) & 0xFFFF)


def _proj_call(x, ws, bs, wd, bd):
    grid = (_N // _BN,)
    full = lambda shape: pl.BlockSpec(shape, lambda i: (0,) * len(shape))
    rows = lambda w: pl.BlockSpec((_BN, w), lambda i: (i, 0))
    return pl.pallas_call(
        _proj_body,
        grid=grid,
        in_specs=[rows(_NDIM), full((_NDIM, _SW)), full((1, _SW)),
                  full((_NDIM, _DW)), full((1, _DW))],
        out_specs=[rows(_SWP), rows(_DWP)],
        out_shape=[jax.ShapeDtypeStruct((_N, _SWP), jnp.int32),
                   jax.ShapeDtypeStruct((_N, _DWP), jnp.int32)],
    )(x, ws, bs, wd, bd)


def _ln_mm(x, ones_mm, g, b):
    m = jnp.dot(x, ones_mm, preferred_element_type=_f32)
    v = jnp.dot(x * x, ones_mm, preferred_element_type=_f32) - m * m
    return (x - m) / jnp.sqrt(v + 1e-5) * g + b


def _unpack_lo(v):
    return lax.bitcast_convert_type(v << 16, _f32)


def _unpack_hi(v):
    return lax.bitcast_convert_type(v & jnp.int32(-65536), _f32)


def _edge_body(use_feat, sg, dg, lg_in, rel,
               wlg, ba, ge1, be1, w1, b1, w2, b2, ge2, be2,
               w_ref, e128_ref, lgn_ref):
    r16 = lax.broadcasted_iota(jnp.int32, (_EDIM, _NDIM), 0)
    c16 = lax.broadcasted_iota(jnp.int32, (_EDIM, _NDIM), 1)
    t16 = (c16 % _EDIM == r16).astype(_f32)
    rs = lax.broadcasted_iota(jnp.int32, (_NDIM, _H), 0)
    cs = lax.broadcasted_iota(jnp.int32, (_NDIM, _H), 1)
    sel = (rs // _DH == cs).astype(_f32)
    rb = lax.broadcasted_iota(jnp.int32, (_H, _NDIM), 0)
    cb = lax.broadcasted_iota(jnp.int32, (_H, _NDIM), 1)
    selt = (cb // _DH == rb).astype(_f32)
    re = lax.broadcasted_iota(jnp.int32, (_NDIM, _EDIM), 0)
    ce = lax.broadcasted_iota(jnp.int32, (_NDIM, _EDIM), 1)
    ext16 = (re == ce).astype(_f32)

    s_i = sg[...]
    d_i = dg[...]
    ksv = s_i[:, :128]
    ks = _unpack_lo(ksv)
    vs = _unpack_hi(ksv)
    xsb = _unpack_lo(s_i[:, 128:256])
    qd = _unpack_lo(d_i)
    xdb = _unpack_hi(d_i)
    xss = jnp.dot(xsb, ext16, preferred_element_type=_f32)
    xdd = jnp.dot(xdb, ext16, preferred_element_type=_f32)

    if use_feat:
        lanes = lax.broadcasted_iota(jnp.int32, (_BE, _R), 1)
        onehot = (lg_in[...] == lanes).astype(_f32)
        lgb = jnp.dot(onehot, rel[...], preferred_element_type=_f32)
    else:
        lgb = lg_in[...]
    lg128 = jnp.dot(lgb, t16, preferred_element_type=_f32)
    t = qd * (ks + lg128)
    esc = jnp.exp(jnp.dot(t, sel, preferred_element_type=_f32) * 0.25)
    i = pl.program_id(0)
    rid = i * _BE + lax.broadcasted_iota(jnp.int32, (_BE, _H), 0)
    esc = jnp.where(rid < _E, esc, 0.0)
    e128 = jnp.dot(esc, selt, preferred_element_type=_f32)
    w_ref[...] = (vs + lg128) * e128
    e128_ref[...] = e128

    one16 = jnp.full((_EDIM, _EDIM), 1.0 / _EDIM, _f32)
    a = jnp.tanh(jnp.dot(lgb, wlg[...], preferred_element_type=_f32)
                 + xss + xdd + ba[...])
    eh = _ln_mm(lgb + a, one16, ge1[...], be1[...])
    ff = jnp.dot(jnp.maximum(jnp.dot(eh, w1[...], preferred_element_type=_f32)
                             + b1[...], 0.0),
                 w2[...], preferred_element_type=_f32) + b2[...]
    lgn_ref[...] = _ln_mm(eh + ff, one16, ge2[...], be2[...])


def _edge_call(use_feat, srcg, dstg, lg_or_feat, rel,
               wlg, ba, ge1, be1, w1, b1, w2, b2, ge2, be2):
    grid = (_EPAD // _BE,)
    full = lambda shape: pl.BlockSpec(shape, lambda i: (0,) * len(shape))
    rows = lambda w: pl.BlockSpec((_BE, w), lambda i: (i, 0))
    lg_spec = (pl.BlockSpec((_BE, 1), lambda i: (i, 0)) if use_feat
               else rows(_EDIM))
    return pl.pallas_call(
        functools.partial(_edge_body, use_feat),
        grid=grid,
        in_specs=[pl.BlockSpec((_BE, _SWP), lambda i: (i, 0)),
                  pl.BlockSpec((_BE, _DWP), lambda i: (i, 0)),
                  lg_spec,
                  full((_R, _EDIM)),
                  full((_EDIM, _EDIM)), full((1, _EDIM)),
                  full((1, _EDIM)), full((1, _EDIM)),
                  full((_EDIM, 4 * _EDIM)), full((1, 4 * _EDIM)),
                  full((4 * _EDIM, _EDIM)), full((1, _EDIM)),
                  full((1, _EDIM)), full((1, _EDIM))],
        out_specs=[rows(_NDIM), rows(_NDIM), rows(_EDIM)],
        out_shape=[jax.ShapeDtypeStruct((_EPAD, _NDIM), _f32),
                   jax.ShapeDtypeStruct((_EPAD, _NDIM), _f32),
                   jax.ShapeDtypeStruct((_EPAD, _EDIM), _f32)],
    )(srcg, dstg, lg_or_feat, rel,
      wlg, ba, ge1, be1, w1, b1, w2, b2, ge2, be2)


def _node_body(x_ref, op, sp, wo, bo, g1, b1, w1, f1, w2, f2, g2, b2,
               out_ref):
    o_un = op[...]
    sv = sp[...]
    o = o_un / (sv + 1e-12)
    o = jnp.dot(o, wo[...], preferred_element_type=_f32) + bo[...]
    one128 = jnp.full((_NDIM, _NDIM), 1.0 / _NDIM, _f32)
    h = _ln_mm(x_ref[...] + o, one128, g1[...], b1[...])
    ff = jnp.dot(jnp.maximum(jnp.dot(h, w1[...], preferred_element_type=_f32)
                             + f1[...], 0.0),
                 w2[...], preferred_element_type=_f32) + f2[...]
    out_ref[...] = _ln_mm(h + ff, one128, g2[...], b2[...])


def _node_call(x, o_part, s_part, wo, bo, g1, b1, w1, f1, w2, f2, g2, b2):
    grid = (_N // _BN,)
    full = lambda shape: pl.BlockSpec(shape, lambda i: (0,) * len(shape))
    return pl.pallas_call(
        _node_body,
        grid=grid,
        in_specs=[pl.BlockSpec((_BN, _NDIM), lambda i: (i, 0)),
                  pl.BlockSpec((_BN, _NDIM), lambda i: (i, 0)),
                  pl.BlockSpec((_BN, _NDIM), lambda i: (i, 0)),
                  full((_NDIM, _NDIM)), full((1, _NDIM)),
                  full((1, _NDIM)), full((1, _NDIM)),
                  full((_NDIM, 4 * _NDIM)), full((1, 4 * _NDIM)),
                  full((4 * _NDIM, _NDIM)), full((1, _NDIM)),
                  full((1, _NDIM)), full((1, _NDIM))],
        out_specs=pl.BlockSpec((_BN, _NDIM), lambda i: (i, 0)),
        out_shape=jax.ShapeDtypeStruct((_N, _NDIM), _f32),
    )(x, o_part, s_part, wo, bo, g1, b1, w1, f1, w2, f2, g2, b2)


def kernel(x, params, edge_feat, src_ids, dst_ids):
    p = params
    pad = _EPAD - _E
    src_p = jnp.pad(src_ids.astype(jnp.int32), (0, pad))
    dst_p = jnp.pad(dst_ids.astype(jnp.int32), (0, pad))
    feat_p = jnp.pad(edge_feat.astype(jnp.int32), (0, pad)).reshape(_EPAD, 1)
    src_g = src_p.reshape(_EPAD // _CH, _CH)
    dst_g = dst_p.reshape(_EPAD // _CH, _CH)
    dst_s = dst_p.reshape(_EPAD // _CHS, _CHS)

    gather = _make_gather()
    scatter = _make_scatter()
    o_zero = jnp.zeros((_NPAD, _NDIM), _f32)

    row = lambda v: v.reshape(1, -1)
    z16 = jnp.zeros((_NDIM, _EDIM), _f32)
    zpad = jnp.zeros((_NDIM, _SW - 2 * _NDIM - _EDIM), _f32)
    zpad_d = jnp.zeros((_NDIM, _DW - _NDIM - _EDIM), _f32)

    lg = jnp.zeros((_EPAD, _EDIM), _f32)
    for l in range(_L):
        w_src = p['aff_W'][l][_EDIM:_EDIM + _NDIM]
        w_dst = p['aff_W'][l][_EDIM + _NDIM:]
        wlg = p['aff_W'][l][:_EDIM]
        ws = jnp.concatenate([p['Wk'][l], p['Wv'][l], w_src, zpad], axis=1)
        bs = jnp.concatenate([p['bk'][l], p['bv'][l],
                              jnp.zeros((_SW - 2 * _NDIM,), _f32)])
        wd = jnp.concatenate([p['Wq'][l], w_dst, zpad_d], axis=1)
        bd = jnp.concatenate([p['bq'][l], jnp.zeros((_DW - _NDIM,), _f32)])
        st, dt = _proj_call(x, ws, row(bs), wd, row(bd))
        srcg, dstg = gather(st, dt, src_g, dst_g)
        use_feat = (l == 0)
        w, e128, lgn = _edge_call(
            use_feat, srcg, dstg, feat_p if use_feat else lg, p['rel_embed'],
            wlg, row(p['aff_b'][l]),
            row(p['ln_e1_g'][l]), row(p['ln_e1_b'][l]),
            p['ffn_e_W1'][l], row(p['ffn_e_b1'][l]),
            p['ffn_e_W2'][l], row(p['ffn_e_b2'][l]),
            row(p['ln_e2_g'][l]), row(p['ln_e2_b'][l]))
        o_part, s_part = scatter(w, e128, dst_s, o_zero)
        x = _node_call(
            x, o_part, s_part, p['Wo'][l], row(p['bo'][l]),
            row(p['ln_n1_g'][l]), row(p['ln_n1_b'][l]),
            p['ffn_n_W1'][l], row(p['ffn_n_b1'][l]),
            p['ffn_n_W2'][l], row(p['ffn_n_b2'][l]),
            row(p['ln_n2_g'][l]), row(p['ln_n2_b'][l]))
        lg = lgn
    return (x, lg[:_E])


# speedup vs baseline: 24.9297x; 1.0130x over previous
import functools

import jax
import jax.numpy as jnp
from jax import lax
from jax.experimental import pallas as pl
from jax.experimental.pallas import tpu as pltpu
from jax.experimental.pallas import tpu_sc as plsc

_N = 10000
_E = 160000
_NDIM = 128
_H = 8
_DH = 16
_EDIM = 16
_L = 2
_R = 128

_NC = 2
_NS = 16
_NW = _NC * _NS

_CH = 128
_NCHUNK = 40
_EW = _CH * _NCHUNK
_EPAD = _EW * _NW

_SW = 384
_DW = 256
_SWP = 256
_DWP = 128

_NPAD = 10240
_CHS = 128
_NCHUNK_S = _EW // _CHS

_f32 = jnp.float32
_bf16 = jnp.bfloat16


def _make_gather():
    widths = [_SWP, _DWP]
    nt = 2
    nb = 2
    mesh = plsc.VectorSubcoreMesh(core_axis_name="c", subcore_axis_name="s")
    out_type = [jax.ShapeDtypeStruct((_EPAD, w), jnp.int32) for w in widths]
    scratch = (
        [pltpu.VMEM((_NCHUNK, _CH), jnp.int32) for _ in widths]
        + [pltpu.VMEM((_CH, w), jnp.int32) for w in widths for _ in range(nb)]
        + [pltpu.SemaphoreType.DMA for _ in widths for _ in range(2 * nb)]
    )

    @functools.partial(pl.kernel, out_type=out_type, mesh=mesh,
                       scratch_types=scratch)
    def gather_kernel(*refs):
        tables = refs[0:nt]
        idxs = refs[nt:2 * nt]
        outs = refs[2 * nt:3 * nt]
        k = 3 * nt
        islabs = refs[k:k + nt]
        k += nt
        rbufs = [refs[k + t * nb:k + (t + 1) * nb] for t in range(nt)]
        k += nt * nb
        sems = [refs[k + t * nb:k + (t + 1) * nb] for t in range(nt)]
        k += nt * nb
        osems = [refs[k + t * nb:k + (t + 1) * nb] for t in range(nt)]
        wid = lax.axis_index("s") * _NC + lax.axis_index("c")
        base = pl.multiple_of(wid * _EW, 8)
        crow = pl.multiple_of(wid * _NCHUNK, 8)

        for t in range(nt):
            pltpu.sync_copy(idxs[t].at[pl.ds(crow, _NCHUNK)], islabs[t])

        def fire(i, b):
            for t in range(nt):
                pltpu.async_copy(tables[t].at[islabs[t].at[i]], rbufs[t][b],
                                 sems[t][b])

        def drain(i, b):
            off = pl.multiple_of(base + i * _CH, 8)
            for t in range(nt):
                pltpu.make_async_copy(tables[t].at[islabs[t].at[i]],
                                      rbufs[t][b], sems[t][b]).wait()
                pltpu.async_copy(rbufs[t][b], outs[t].at[pl.ds(off, _CH)],
                                 osems[t][b])

        def wait_out(i, b):
            off = pl.multiple_of(base + i * _CH, 8)
            for t in range(nt):
                pltpu.make_async_copy(rbufs[t][b],
                                      outs[t].at[pl.ds(off, _CH)],
                                      osems[t][b]).wait()

        for i in range(nb):
            fire(i, i)

        def body(j, carry):
            i0 = j * nb
            for b in range(nb):
                i = i0 + b
                drain(i, b)

                @pl.when(i + nb < _NCHUNK)
                def _next():
                    wait_out(i, b)
                    fire(i + nb, b)
            return carry

        lax.fori_loop(0, _NCHUNK // nb, body, 0)
        for b in range(nb):
            wait_out(_NCHUNK - nb + b, b)

    return gather_kernel


_EW2 = _EPAD // _NS
_NCH2 = _EW2 // _CHS


def _make_scatter():
    nb = 2
    mesh = plsc.VectorSubcoreMesh(core_axis_name="c", subcore_axis_name="s")
    out_type = [
        jax.ShapeDtypeStruct((_NPAD, _NDIM), _f32),
        jax.ShapeDtypeStruct((_NPAD, _NDIM), _f32),
    ]
    scratch = (
        [pltpu.VMEM_SHARED((_NPAD, _NDIM), _f32),
         pltpu.VMEM((_NCH2, _CHS), jnp.int32)]
        + [pltpu.VMEM((_CHS, _NDIM), _f32) for _ in range(nb)]
        + [pltpu.SemaphoreType.DMA for _ in range(2 * nb)]
    )

    @functools.partial(pl.kernel, out_type=out_type, mesh=mesh,
                       scratch_types=scratch)
    def scatter_kernel(w_hbm, e_hbm, dst_hbm, oz_hbm, o_out, s_out,
                       acc, islab, wb0, wb1, ls0, ls1, as0, as1):
        c = lax.axis_index("c")
        s = lax.axis_index("s")
        wbs = [wb0, wb1]
        lsems = [ls0, ls1]
        asems = [as0, as1]

        @pl.when(s == 0)
        def _init():
            pltpu.sync_copy(oz_hbm, acc)

        crow = pl.multiple_of(s * _NCH2, 8)
        pltpu.sync_copy(dst_hbm.at[pl.ds(crow, _NCH2)], islab)
        plsc.subcore_barrier()

        base = pl.multiple_of(s * _EW2, 8)

        def run(arr_hbm):
            def fire_load(i, b):
                off = pl.multiple_of(base + i * _CHS, 8)
                pltpu.async_copy(arr_hbm.at[pl.ds(off, _CHS)], wbs[b],
                                 lsems[b])

            def step(i, b):
                off = pl.multiple_of(base + i * _CHS, 8)
                pltpu.make_async_copy(arr_hbm.at[pl.ds(off, _CHS)], wbs[b],
                                      lsems[b]).wait()
                pltpu.async_copy(wbs[b], acc.at[islab.at[i]], asems[b],
                                 add=True)

                @pl.when(i + nb < _NCH2)
                def _next():
                    pltpu.make_async_copy(wbs[b], acc.at[islab.at[i]],
                                          asems[b]).wait()
                    fire_load(i + nb, b)

            for i in range(nb):
                fire_load(i, i)

            def body(j, carry):
                for b in range(nb):
                    step(j * nb + b, b)
                return carry

            lax.fori_loop(0, _NCH2 // nb, body, 0)
            for b in range(nb):
                pltpu.make_async_copy(wbs[b], acc.at[islab.at[_NCH2 - nb + b]],
                                      asems[b]).wait()

        @pl.when(c == 0)
        def _scatter_w():
            run(w_hbm)

        @pl.when(c == 1)
        def _scatter_e():
            run(e_hbm)

        plsc.subcore_barrier()

        @pl.when((s == 0) & (c == 0))
        def _write_o():
            pltpu.sync_copy(acc, o_out)

        @pl.when((s == 0) & (c == 1))
        def _write_s():
            pltpu.sync_copy(acc, s_out)

    return scatter_kernel


_BN = 1000
_BE = 2048


def _bits16(x):
    y = x.astype(_bf16).astype(_f32)
    return lax.bitcast_convert_type(y, jnp.int32)


def _proj_body(x_ref, ws, bs, wd, bd, st_ref, dt_ref):
    x = x_ref[...]
    accs = jnp.dot(x, ws[...], preferred_element_type=_f32) + bs[...]
    accd = jnp.dot(x, wd[...], preferred_element_type=_f32) + bd[...]
    kp, vp, xsb = accs[:, :128], accs[:, 128:256], accs[:, 256:384]
    qp, xdb = accd[:, :128], accd[:, 128:256]
    hi_mask = jnp.int32(-65536)
    pk1 = (_bits16(vp) & hi_mask) | ((_bits16(kp) >> 16) & 0xFFFF)
    pk2 = (_bits16(xsb) >> 16) & 0xFFFF
    st_ref[...] = jnp.concatenate([pk1, pk2], axis=1)
    dt_ref[...] = (_bits16(xdb) & hi_mask) | ((_bits16(qp) >> 16) & 0xFFFF)


def _proj_call(x, ws, bs, wd, bd):
    grid = (_N // _BN,)
    full = lambda shape: pl.BlockSpec(shape, lambda i: (0,) * len(shape))
    rows = lambda w: pl.BlockSpec((_BN, w), lambda i: (i, 0))
    return pl.pallas_call(
        _proj_body,
        grid=grid,
        in_specs=[rows(_NDIM), full((_NDIM, _SW)), full((1, _SW)),
                  full((_NDIM, _DW)), full((1, _DW))],
        out_specs=[rows(_SWP), rows(_DWP)],
        out_shape=[jax.ShapeDtypeStruct((_N, _SWP), jnp.int32),
                   jax.ShapeDtypeStruct((_N, _DWP), jnp.int32)],
    )(x, ws, bs, wd, bd)


def _ln_mm(x, ones_mm, g, b):
    m = jnp.dot(x, ones_mm, preferred_element_type=_f32)
    v = jnp.dot(x * x, ones_mm, preferred_element_type=_f32) - m * m
    return (x - m) / jnp.sqrt(v + 1e-5) * g + b


def _unpack_lo(v):
    return lax.bitcast_convert_type(v << 16, _f32)


def _unpack_hi(v):
    return lax.bitcast_convert_type(v & jnp.int32(-65536), _f32)


def _edge_body(use_feat, sg, dg, lg_in, rel,
               wlg, ba, ge1, be1, w1, b1, w2, b2, ge2, be2,
               w_ref, e128_ref, lgn_ref):
    r16 = lax.broadcasted_iota(jnp.int32, (_EDIM, _NDIM), 0)
    c16 = lax.broadcasted_iota(jnp.int32, (_EDIM, _NDIM), 1)
    t16 = (c16 % _EDIM == r16).astype(_f32)
    rs = lax.broadcasted_iota(jnp.int32, (_NDIM, _H), 0)
    cs = lax.broadcasted_iota(jnp.int32, (_NDIM, _H), 1)
    sel = (rs // _DH == cs).astype(_f32)
    rb = lax.broadcasted_iota(jnp.int32, (_H, _NDIM), 0)
    cb = lax.broadcasted_iota(jnp.int32, (_H, _NDIM), 1)
    selt = (cb // _DH == rb).astype(_f32)
    re = lax.broadcasted_iota(jnp.int32, (_NDIM, _EDIM), 0)
    ce = lax.broadcasted_iota(jnp.int32, (_NDIM, _EDIM), 1)
    ext16 = (re == ce).astype(_f32)

    s_i = sg[...]
    d_i = dg[...]
    ksv = s_i[:, :128]
    ks = _unpack_lo(ksv)
    vs = _unpack_hi(ksv)
    xsb = _unpack_lo(s_i[:, 128:256])
    qd = _unpack_lo(d_i)
    xdb = _unpack_hi(d_i)
    xss = jnp.dot(xsb, ext16, preferred_element_type=_f32)
    xdd = jnp.dot(xdb, ext16, preferred_element_type=_f32)

    if use_feat:
        lanes = lax.broadcasted_iota(jnp.int32, (_BE, _R), 1)
        onehot = (lg_in[...] == lanes).astype(_f32)
        lgb = jnp.dot(onehot, rel[...], preferred_element_type=_f32)
    else:
        lgb = lg_in[...]
    lg128 = jnp.dot(lgb, t16, preferred_element_type=_f32)
    t = qd * (ks + lg128)
    esc = jnp.exp(jnp.dot(t, sel, preferred_element_type=_f32) * 0.25)
    i = pl.program_id(0)
    rid = i * _BE + lax.broadcasted_iota(jnp.int32, (_BE, _H), 0)
    esc = jnp.where(rid < _E, esc, 0.0)
    e128 = jnp.dot(esc, selt, preferred_element_type=_f32)
    w_ref[...] = (vs + lg128) * e128
    e128_ref[...] = e128

    one16 = jnp.full((_EDIM, _EDIM), 1.0 / _EDIM, _f32)
    a = jnp.tanh(jnp.dot(lgb, wlg[...], preferred_element_type=_f32)
                 + xss + xdd + ba[...])
    eh = _ln_mm(lgb + a, one16, ge1[...], be1[...])
    ff = jnp.dot(jnp.maximum(jnp.dot(eh, w1[...], preferred_element_type=_f32)
                             + b1[...], 0.0),
                 w2[...], preferred_element_type=_f32) + b2[...]
    lgn_ref[...] = _ln_mm(eh + ff, one16, ge2[...], be2[...])


def _edge_call(use_feat, srcg, dstg, lg_or_feat, rel,
               wlg, ba, ge1, be1, w1, b1, w2, b2, ge2, be2):
    grid = (_EPAD // _BE,)
    full = lambda shape: pl.BlockSpec(shape, lambda i: (0,) * len(shape))
    rows = lambda w: pl.BlockSpec((_BE, w), lambda i: (i, 0))
    lg_spec = (pl.BlockSpec((_BE, 1), lambda i: (i, 0)) if use_feat
               else rows(_EDIM))
    return pl.pallas_call(
        functools.partial(_edge_body, use_feat),
        grid=grid,
        in_specs=[pl.BlockSpec((_BE, _SWP), lambda i: (i, 0)),
                  pl.BlockSpec((_BE, _DWP), lambda i: (i, 0)),
                  lg_spec,
                  full((_R, _EDIM)),
                  full((_EDIM, _EDIM)), full((1, _EDIM)),
                  full((1, _EDIM)), full((1, _EDIM)),
                  full((_EDIM, 4 * _EDIM)), full((1, 4 * _EDIM)),
                  full((4 * _EDIM, _EDIM)), full((1, _EDIM)),
                  full((1, _EDIM)), full((1, _EDIM))],
        out_specs=[rows(_NDIM), rows(_NDIM), rows(_EDIM)],
        out_shape=[jax.ShapeDtypeStruct((_EPAD, _NDIM), _f32),
                   jax.ShapeDtypeStruct((_EPAD, _NDIM), _f32),
                   jax.ShapeDtypeStruct((_EPAD, _EDIM), _f32)],
    )(srcg, dstg, lg_or_feat, rel,
      wlg, ba, ge1, be1, w1, b1, w2, b2, ge2, be2)


def _node_body(x_ref, op, sp, wo, bo, g1, b1, w1, f1, w2, f2, g2, b2,
               out_ref):
    o_un = op[...]
    sv = sp[...]
    o = o_un / (sv + 1e-12)
    o = jnp.dot(o, wo[...], preferred_element_type=_f32) + bo[...]
    one128 = jnp.full((_NDIM, _NDIM), 1.0 / _NDIM, _f32)
    h = _ln_mm(x_ref[...] + o, one128, g1[...], b1[...])
    ff = jnp.dot(jnp.maximum(jnp.dot(h, w1[...], preferred_element_type=_f32)
                             + f1[...], 0.0),
                 w2[...], preferred_element_type=_f32) + f2[...]
    out_ref[...] = _ln_mm(h + ff, one128, g2[...], b2[...])


def _node_call(x, o_part, s_part, wo, bo, g1, b1, w1, f1, w2, f2, g2, b2):
    grid = (_N // _BN,)
    full = lambda shape: pl.BlockSpec(shape, lambda i: (0,) * len(shape))
    return pl.pallas_call(
        _node_body,
        grid=grid,
        in_specs=[pl.BlockSpec((_BN, _NDIM), lambda i: (i, 0)),
                  pl.BlockSpec((_BN, _NDIM), lambda i: (i, 0)),
                  pl.BlockSpec((_BN, _NDIM), lambda i: (i, 0)),
                  full((_NDIM, _NDIM)), full((1, _NDIM)),
                  full((1, _NDIM)), full((1, _NDIM)),
                  full((_NDIM, 4 * _NDIM)), full((1, 4 * _NDIM)),
                  full((4 * _NDIM, _NDIM)), full((1, _NDIM)),
                  full((1, _NDIM)), full((1, _NDIM))],
        out_specs=pl.BlockSpec((_BN, _NDIM), lambda i: (i, 0)),
        out_shape=jax.ShapeDtypeStruct((_N, _NDIM), _f32),
    )(x, o_part, s_part, wo, bo, g1, b1, w1, f1, w2, f2, g2, b2)


def kernel(x, params, edge_feat, src_ids, dst_ids):
    p = params
    pad = _EPAD - _E
    src_p = jnp.pad(src_ids.astype(jnp.int32), (0, pad))
    dst_p = jnp.pad(dst_ids.astype(jnp.int32), (0, pad))
    feat_p = jnp.pad(edge_feat.astype(jnp.int32), (0, pad)).reshape(_EPAD, 1)
    src_g = src_p.reshape(_EPAD // _CH, _CH)
    dst_g = dst_p.reshape(_EPAD // _CH, _CH)
    dst_s = dst_p.reshape(_EPAD // _CHS, _CHS)

    gather = _make_gather()
    scatter = _make_scatter()
    o_zero = jnp.zeros((_NPAD, _NDIM), _f32)

    row = lambda v: v.reshape(1, -1)
    z16 = jnp.zeros((_NDIM, _EDIM), _f32)
    zpad = jnp.zeros((_NDIM, _SW - 2 * _NDIM - _EDIM), _f32)
    zpad_d = jnp.zeros((_NDIM, _DW - _NDIM - _EDIM), _f32)

    lg = jnp.zeros((_EPAD, _EDIM), _f32)
    for l in range(_L):
        w_src = p['aff_W'][l][_EDIM:_EDIM + _NDIM]
        w_dst = p['aff_W'][l][_EDIM + _NDIM:]
        wlg = p['aff_W'][l][:_EDIM]
        ws = jnp.concatenate([p['Wk'][l], p['Wv'][l], w_src, zpad], axis=1)
        bs = jnp.concatenate([p['bk'][l], p['bv'][l],
                              jnp.zeros((_SW - 2 * _NDIM,), _f32)])
        wd = jnp.concatenate([p['Wq'][l], w_dst, zpad_d], axis=1)
        bd = jnp.concatenate([p['bq'][l], jnp.zeros((_DW - _NDIM,), _f32)])
        st, dt = _proj_call(x, ws, row(bs), wd, row(bd))
        srcg, dstg = gather(st, dt, src_g, dst_g)
        use_feat = (l == 0)
        w, e128, lgn = _edge_call(
            use_feat, srcg, dstg, feat_p if use_feat else lg, p['rel_embed'],
            wlg, row(p['aff_b'][l]),
            row(p['ln_e1_g'][l]), row(p['ln_e1_b'][l]),
            p['ffn_e_W1'][l], row(p['ffn_e_b1'][l]),
            p['ffn_e_W2'][l], row(p['ffn_e_b2'][l]),
            row(p['ln_e2_g'][l]), row(p['ln_e2_b'][l]))
        o_part, s_part = scatter(w, e128, dst_s, o_zero)
        x = _node_call(
            x, o_part, s_part, p['Wo'][l], row(p['bo'][l]),
            row(p['ln_n1_g'][l]), row(p['ln_n1_b'][l]),
            p['ffn_n_W1'][l], row(p['ffn_n_b1'][l]),
            p['ffn_n_W2'][l], row(p['ffn_n_b2'][l]),
            row(p['ln_n2_g'][l]), row(p['ln_n2_b'][l]))
        lg = lgn
    return (x, lg[:_E])


# speedup vs baseline: 27.3910x; 1.0987x over previous
import functools

import jax
import jax.numpy as jnp
from jax import lax
from jax.experimental import pallas as pl
from jax.experimental.pallas import tpu as pltpu
from jax.experimental.pallas import tpu_sc as plsc

_N = 10000
_E = 160000
_NDIM = 128
_H = 8
_DH = 16
_EDIM = 16
_L = 2
_R = 128

_NC = 2
_NS = 16
_NW = _NC * _NS

_CH = 128
_NCHUNK = 40
_EW = _CH * _NCHUNK
_EPAD = _EW * _NW

_SW = 384
_DW = 256
_SWP = 256
_DWP = 128

_NPAD = 10240
_CHS = 128
_NCHUNK_S = _EW // _CHS

_f32 = jnp.float32
_bf16 = jnp.bfloat16


def _make_gather():
    epad, ch = _EPAD // 2, 64
    ew = epad // _NW
    nchunk = ew // ch
    widths = [_SWP, _DWP]
    nt = 2
    nb = 2
    mesh = plsc.VectorSubcoreMesh(core_axis_name="c", subcore_axis_name="s")
    out_type = [jax.ShapeDtypeStruct((epad, w), jnp.int32) for w in widths]
    scratch = (
        [pltpu.VMEM((nchunk, ch), jnp.int32) for _ in widths]
        + [pltpu.VMEM((ch, w), jnp.int32) for w in widths for _ in range(nb)]
        + [pltpu.SemaphoreType.DMA for _ in widths for _ in range(2 * nb)]
    )

    @functools.partial(pl.kernel, out_type=out_type, mesh=mesh,
                       scratch_types=scratch)
    def gather_kernel(*refs):
        tables = refs[0:nt]
        idxs = refs[nt:2 * nt]
        outs = refs[2 * nt:3 * nt]
        k = 3 * nt
        islabs = refs[k:k + nt]
        k += nt
        rbufs = [refs[k + t * nb:k + (t + 1) * nb] for t in range(nt)]
        k += nt * nb
        sems = [refs[k + t * nb:k + (t + 1) * nb] for t in range(nt)]
        k += nt * nb
        osems = [refs[k + t * nb:k + (t + 1) * nb] for t in range(nt)]
        wid = lax.axis_index("s") * _NC + lax.axis_index("c")
        base = pl.multiple_of(wid * ew, 8)
        crow = pl.multiple_of(wid * nchunk, 8)

        for t in range(nt):
            pltpu.sync_copy(idxs[t].at[pl.ds(crow, nchunk)], islabs[t])

        def fire(i, b):
            for t in range(nt):
                pltpu.async_copy(tables[t].at[islabs[t].at[i]], rbufs[t][b],
                                 sems[t][b])

        def drain(i, b):
            off = pl.multiple_of(base + i * ch, 8)
            for t in range(nt):
                pltpu.make_async_copy(tables[t].at[islabs[t].at[i]],
                                      rbufs[t][b], sems[t][b]).wait()
                pltpu.async_copy(rbufs[t][b], outs[t].at[pl.ds(off, ch)],
                                 osems[t][b])

        def wait_out(i, b):
            off = pl.multiple_of(base + i * ch, 8)
            for t in range(nt):
                pltpu.make_async_copy(rbufs[t][b],
                                      outs[t].at[pl.ds(off, ch)],
                                      osems[t][b]).wait()

        for i in range(nb):
            fire(i, i)

        def body(j, carry):
            i0 = j * nb
            for b in range(nb):
                i = i0 + b
                drain(i, b)

                @pl.when(i + nb < nchunk)
                def _next():
                    wait_out(i, b)
                    fire(i + nb, b)
            return carry

        lax.fori_loop(0, nchunk // nb, body, 0)
        for b in range(nb):
            wait_out(nchunk - nb + b, b)

    return gather_kernel


_EW2 = _EPAD // _NS
_NCH2 = _EW2 // _CHS


def _make_scatter():
    nb = 2
    mesh = plsc.VectorSubcoreMesh(core_axis_name="c", subcore_axis_name="s")
    out_type = [
        jax.ShapeDtypeStruct((_NPAD, _NDIM), _f32),
        jax.ShapeDtypeStruct((_NPAD, _NDIM), _f32),
    ]
    scratch = (
        [pltpu.VMEM_SHARED((_NPAD, _NDIM), _f32),
         pltpu.VMEM((_NCH2, _CHS), jnp.int32)]
        + [pltpu.VMEM((_CHS, _NDIM), _f32) for _ in range(nb)]
        + [pltpu.SemaphoreType.DMA for _ in range(2 * nb)]
    )

    @functools.partial(pl.kernel, out_type=out_type, mesh=mesh,
                       scratch_types=scratch)
    def scatter_kernel(w0_hbm, w1_hbm, e0_hbm, e1_hbm, dst_hbm, oz_hbm,
                       o_out, s_out,
                       acc, islab, wb0, wb1, ls0, ls1, as0, as1):
        c = lax.axis_index("c")
        s = lax.axis_index("s")
        wbs = [wb0, wb1]
        lsems = [ls0, ls1]
        asems = [as0, as1]

        @pl.when(s == 0)
        def _init():
            pltpu.sync_copy(oz_hbm, acc)

        crow = pl.multiple_of(s * _NCH2, 8)
        pltpu.sync_copy(dst_hbm.at[pl.ds(crow, _NCH2)], islab)
        plsc.subcore_barrier()

        def run(arr_hbm, sl):
            base = pl.multiple_of(sl * _EW2, 8)

            def fire_load(i, b):
                off = pl.multiple_of(base + i * _CHS, 8)
                pltpu.async_copy(arr_hbm.at[pl.ds(off, _CHS)], wbs[b],
                                 lsems[b])

            def step(i, b):
                off = pl.multiple_of(base + i * _CHS, 8)
                pltpu.make_async_copy(arr_hbm.at[pl.ds(off, _CHS)], wbs[b],
                                      lsems[b]).wait()
                pltpu.async_copy(wbs[b], acc.at[islab.at[i]], asems[b],
                                 add=True)

                @pl.when(i + nb < _NCH2)
                def _next():
                    pltpu.make_async_copy(wbs[b], acc.at[islab.at[i]],
                                          asems[b]).wait()
                    fire_load(i + nb, b)

            for i in range(nb):
                fire_load(i, i)

            def body(j, carry):
                for b in range(nb):
                    step(j * nb + b, b)
                return carry

            lax.fori_loop(0, _NCH2 // nb, body, 0)
            for b in range(nb):
                pltpu.make_async_copy(wbs[b], acc.at[islab.at[_NCH2 - nb + b]],
                                      asems[b]).wait()

        half = _NS // 2

        @pl.when((c == 0) & (s < half))
        def _scatter_w0():
            run(w0_hbm, s)

        @pl.when((c == 0) & (s >= half))
        def _scatter_w1():
            run(w1_hbm, s - half)

        @pl.when((c == 1) & (s < half))
        def _scatter_e0():
            run(e0_hbm, s)

        @pl.when((c == 1) & (s >= half))
        def _scatter_e1():
            run(e1_hbm, s - half)

        plsc.subcore_barrier()

        @pl.when((s == 0) & (c == 0))
        def _write_o():
            pltpu.sync_copy(acc, o_out)

        @pl.when((s == 0) & (c == 1))
        def _write_s():
            pltpu.sync_copy(acc, s_out)

    return scatter_kernel


_BN = 1000
_BE = 2048


def _bits16(x):
    y = x.astype(_bf16).astype(_f32)
    return lax.bitcast_convert_type(y, jnp.int32)


def _proj_body(x_ref, ws, bs, wd, bd, st_ref, dt_ref):
    x = x_ref[...]
    accs = jnp.dot(x, ws[...], preferred_element_type=_f32) + bs[...]
    accd = jnp.dot(x, wd[...], preferred_element_type=_f32) + bd[...]
    kp, vp, xsb = accs[:, :128], accs[:, 128:256], accs[:, 256:384]
    qp, xdb = accd[:, :128], accd[:, 128:256]
    hi_mask = jnp.int32(-65536)
    pk1 = (_bits16(vp) & hi_mask) | ((_bits16(kp) >> 16) & 0xFFFF)
    pk2 = (_bits16(xsb) >> 16) & 0xFFFF
    st_ref[...] = jnp.concatenate([pk1, pk2], axis=1)
    dt_ref[...] = (_bits16(xdb) & hi_mask) | ((_bits16(qp) >> 16) & 0xFFFF)


def _proj_call(x, ws, bs, wd, bd):
    grid = (_N // _BN,)
    full = lambda shape: pl.BlockSpec(shape, lambda i: (0,) * len(shape))
    rows = lambda w: pl.BlockSpec((_BN, w), lambda i: (i, 0))
    return pl.pallas_call(
        _proj_body,
        grid=grid,
        in_specs=[rows(_NDIM), full((_NDIM, _SW)), full((1, _SW)),
                  full((_NDIM, _DW)), full((1, _DW))],
        out_specs=[rows(_SWP), rows(_DWP)],
        out_shape=[jax.ShapeDtypeStruct((_N, _SWP), jnp.int32),
                   jax.ShapeDtypeStruct((_N, _DWP), jnp.int32)],
    )(x, ws, bs, wd, bd)


def _ln_mm(x, ones_mm, g, b):
    m = jnp.dot(x, ones_mm, preferred_element_type=_f32)
    v = jnp.dot(x * x, ones_mm, preferred_element_type=_f32) - m * m
    return (x - m) / jnp.sqrt(v + 1e-5) * g + b


def _unpack_lo(v):
    return lax.bitcast_convert_type(v << 16, _f32)


def _unpack_hi(v):
    return lax.bitcast_convert_type(v & jnp.int32(-65536), _f32)


def _edge_body(use_feat, row0, sg, dg, lg_in, rel,
               wlg, ba, ge1, be1, w1, b1, w2, b2, ge2, be2,
               w_ref, e128_ref, lgn_ref):
    r16 = lax.broadcasted_iota(jnp.int32, (_EDIM, _NDIM), 0)
    c16 = lax.broadcasted_iota(jnp.int32, (_EDIM, _NDIM), 1)
    t16 = (c16 % _EDIM == r16).astype(_f32)
    rs = lax.broadcasted_iota(jnp.int32, (_NDIM, _H), 0)
    cs = lax.broadcasted_iota(jnp.int32, (_NDIM, _H), 1)
    sel = (rs // _DH == cs).astype(_f32)
    rb = lax.broadcasted_iota(jnp.int32, (_H, _NDIM), 0)
    cb = lax.broadcasted_iota(jnp.int32, (_H, _NDIM), 1)
    selt = (cb // _DH == rb).astype(_f32)
    re = lax.broadcasted_iota(jnp.int32, (_NDIM, _EDIM), 0)
    ce = lax.broadcasted_iota(jnp.int32, (_NDIM, _EDIM), 1)
    ext16 = (re == ce).astype(_f32)

    s_i = sg[...]
    d_i = dg[...]
    ksv = s_i[:, :128]
    ks = _unpack_lo(ksv)
    vs = _unpack_hi(ksv)
    xsb = _unpack_lo(s_i[:, 128:256])
    qd = _unpack_lo(d_i)
    xdb = _unpack_hi(d_i)
    xss = jnp.dot(xsb, ext16, preferred_element_type=_f32)
    xdd = jnp.dot(xdb, ext16, preferred_element_type=_f32)

    if use_feat:
        lanes = lax.broadcasted_iota(jnp.int32, (_BE, _R), 1)
        onehot = (lg_in[...] == lanes).astype(_f32)
        lgb = jnp.dot(onehot, rel[...], preferred_element_type=_f32)
    else:
        lgb = lg_in[...]
    lg128 = jnp.dot(lgb, t16, preferred_element_type=_f32)
    t = qd * (ks + lg128)
    esc = jnp.exp(jnp.dot(t, sel, preferred_element_type=_f32) * 0.25)
    i = pl.program_id(0)
    rid = row0 + i * _BE + lax.broadcasted_iota(jnp.int32, (_BE, _H), 0)
    esc = jnp.where(rid < _E, esc, 0.0)
    e128 = jnp.dot(esc, selt, preferred_element_type=_f32)
    w_ref[...] = (vs + lg128) * e128
    e128_ref[...] = e128

    one16 = jnp.full((_EDIM, _EDIM), 1.0 / _EDIM, _f32)
    a = jnp.tanh(jnp.dot(lgb, wlg[...], preferred_element_type=_f32)
                 + xss + xdd + ba[...])
    eh = _ln_mm(lgb + a, one16, ge1[...], be1[...])
    ff = jnp.dot(jnp.maximum(jnp.dot(eh, w1[...], preferred_element_type=_f32)
                             + b1[...], 0.0),
                 w2[...], preferred_element_type=_f32) + b2[...]
    lgn_ref[...] = _ln_mm(eh + ff, one16, ge2[...], be2[...])


def _edge_call(use_feat, half, srcg, dstg, lg_or_feat, rel,
               wlg, ba, ge1, be1, w1, b1, w2, b2, ge2, be2):
    hpad = _EPAD // 2
    grid = (hpad // _BE,)
    full = lambda shape: pl.BlockSpec(shape, lambda i: (0,) * len(shape))
    rows = lambda w: pl.BlockSpec((_BE, w), lambda i: (i, 0))
    lg_spec = (pl.BlockSpec((_BE, 1), lambda i: (i, 0)) if use_feat
               else rows(_EDIM))
    return pl.pallas_call(
        functools.partial(_edge_body, use_feat, half * hpad),
        grid=grid,
        in_specs=[pl.BlockSpec((_BE, _SWP), lambda i: (i, 0)),
                  pl.BlockSpec((_BE, _DWP), lambda i: (i, 0)),
                  lg_spec,
                  full((_R, _EDIM)),
                  full((_EDIM, _EDIM)), full((1, _EDIM)),
                  full((1, _EDIM)), full((1, _EDIM)),
                  full((_EDIM, 4 * _EDIM)), full((1, 4 * _EDIM)),
                  full((4 * _EDIM, _EDIM)), full((1, _EDIM)),
                  full((1, _EDIM)), full((1, _EDIM))],
        out_specs=[rows(_NDIM), rows(_NDIM), rows(_EDIM)],
        out_shape=[jax.ShapeDtypeStruct((hpad, _NDIM), _f32),
                   jax.ShapeDtypeStruct((hpad, _NDIM), _f32),
                   jax.ShapeDtypeStruct((hpad, _EDIM), _f32)],
    )(srcg, dstg, lg_or_feat, rel,
      wlg, ba, ge1, be1, w1, b1, w2, b2, ge2, be2)


def _node_body(x_ref, op, sp, wo, bo, g1, b1, w1, f1, w2, f2, g2, b2,
               out_ref):
    o_un = op[...]
    sv = sp[...]
    o = o_un / (sv + 1e-12)
    o = jnp.dot(o, wo[...], preferred_element_type=_f32) + bo[...]
    one128 = jnp.full((_NDIM, _NDIM), 1.0 / _NDIM, _f32)
    h = _ln_mm(x_ref[...] + o, one128, g1[...], b1[...])
    ff = jnp.dot(jnp.maximum(jnp.dot(h, w1[...], preferred_element_type=_f32)
                             + f1[...], 0.0),
                 w2[...], preferred_element_type=_f32) + f2[...]
    out_ref[...] = _ln_mm(h + ff, one128, g2[...], b2[...])


def _node_call(x, o_part, s_part, wo, bo, g1, b1, w1, f1, w2, f2, g2, b2):
    grid = (_N // _BN,)
    full = lambda shape: pl.BlockSpec(shape, lambda i: (0,) * len(shape))
    return pl.pallas_call(
        _node_body,
        grid=grid,
        in_specs=[pl.BlockSpec((_BN, _NDIM), lambda i: (i, 0)),
                  pl.BlockSpec((_BN, _NDIM), lambda i: (i, 0)),
                  pl.BlockSpec((_BN, _NDIM), lambda i: (i, 0)),
                  full((_NDIM, _NDIM)), full((1, _NDIM)),
                  full((1, _NDIM)), full((1, _NDIM)),
                  full((_NDIM, 4 * _NDIM)), full((1, 4 * _NDIM)),
                  full((4 * _NDIM, _NDIM)), full((1, _NDIM)),
                  full((1, _NDIM)), full((1, _NDIM))],
        out_specs=pl.BlockSpec((_BN, _NDIM), lambda i: (i, 0)),
        out_shape=jax.ShapeDtypeStruct((_N, _NDIM), _f32),
    )(x, o_part, s_part, wo, bo, g1, b1, w1, f1, w2, f2, g2, b2)


def kernel(x, params, edge_feat, src_ids, dst_ids):
    p = params
    pad = _EPAD - _E
    hpad = _EPAD // 2
    src_p = jnp.pad(src_ids.astype(jnp.int32), (0, pad))
    dst_p = jnp.pad(dst_ids.astype(jnp.int32), (0, pad))
    feat_p = jnp.pad(edge_feat.astype(jnp.int32), (0, pad))
    feat_h = [feat_p[:hpad].reshape(hpad, 1), feat_p[hpad:].reshape(hpad, 1)]
    src_h = [src_p[:hpad].reshape(hpad // 64, 64),
             src_p[hpad:].reshape(hpad // 64, 64)]
    dst_h = [dst_p[:hpad].reshape(hpad // 64, 64),
             dst_p[hpad:].reshape(hpad // 64, 64)]
    dst_s = dst_p.reshape(_EPAD // _CHS, _CHS)

    gather = _make_gather()
    scatter = _make_scatter()
    o_zero = jnp.zeros((_NPAD, _NDIM), _f32)

    row = lambda v: v.reshape(1, -1)
    z16 = jnp.zeros((_NDIM, _EDIM), _f32)
    zpad = jnp.zeros((_NDIM, _SW - 2 * _NDIM - _EDIM), _f32)
    zpad_d = jnp.zeros((_NDIM, _DW - _NDIM - _EDIM), _f32)

    lg_h = [None, None]
    for l in range(_L):
        w_src = p['aff_W'][l][_EDIM:_EDIM + _NDIM]
        w_dst = p['aff_W'][l][_EDIM + _NDIM:]
        wlg = p['aff_W'][l][:_EDIM]
        ws = jnp.concatenate([p['Wk'][l], p['Wv'][l], w_src, zpad], axis=1)
        bs = jnp.concatenate([p['bk'][l], p['bv'][l],
                              jnp.zeros((_SW - 2 * _NDIM,), _f32)])
        wd = jnp.concatenate([p['Wq'][l], w_dst, zpad_d], axis=1)
        bd = jnp.concatenate([p['bq'][l], jnp.zeros((_DW - _NDIM,), _f32)])
        st, dt = _proj_call(x, ws, row(bs), wd, row(bd))
        use_feat = (l == 0)
        w_h, e_h, lgn_h = [], [], []
        for h in range(2):
            srcg, dstg = gather(st, dt, src_h[h], dst_h[h])
            w, e128, lgn = _edge_call(
                use_feat, h, srcg, dstg,
                feat_h[h] if use_feat else lg_h[h], p['rel_embed'],
                wlg, row(p['aff_b'][l]),
                row(p['ln_e1_g'][l]), row(p['ln_e1_b'][l]),
                p['ffn_e_W1'][l], row(p['ffn_e_b1'][l]),
                p['ffn_e_W2'][l], row(p['ffn_e_b2'][l]),
                row(p['ln_e2_g'][l]), row(p['ln_e2_b'][l]))
            w_h.append(w)
            e_h.append(e128)
            lgn_h.append(lgn)
        o_part, s_part = scatter(w_h[0], w_h[1], e_h[0], e_h[1], dst_s,
                                 o_zero)
        x = _node_call(
            x, o_part, s_part, p['Wo'][l], row(p['bo'][l]),
            row(p['ln_n1_g'][l]), row(p['ln_n1_b'][l]),
            p['ffn_n_W1'][l], row(p['ffn_n_b1'][l]),
            p['ffn_n_W2'][l], row(p['ffn_n_b2'][l]),
            row(p['ln_n2_g'][l]), row(p['ln_n2_b'][l]))
        lg_h = lgn_h
    return (x, jnp.concatenate(lg_h, axis=0)[:_E])


# speedup vs baseline: 27.4611x; 1.0026x over previous
import functools

import jax
import jax.numpy as jnp
from jax import lax
from jax.experimental import pallas as pl
from jax.experimental.pallas import tpu as pltpu
from jax.experimental.pallas import tpu_sc as plsc

_N = 10000
_E = 160000
_NDIM = 128
_H = 8
_DH = 16
_EDIM = 16
_L = 2
_R = 128

_NC = 2
_NS = 16
_NW = _NC * _NS

_EW = 5120
_EPAD = _EW * _NW

_SW = 384
_DW = 256
_SWP = 256
_DWP = 128

_NPAD = 10240
_CHS = 128

_f32 = jnp.float32
_bf16 = jnp.bfloat16


def _make_gather():
    epad, ch = _EPAD // 2, 64
    ew = epad // _NW
    nchunk = ew // ch
    widths = [_SWP, _DWP]
    nt = 2
    nb = 2
    mesh = plsc.VectorSubcoreMesh(core_axis_name="c", subcore_axis_name="s")
    out_type = [jax.ShapeDtypeStruct((epad, w), jnp.int32) for w in widths]
    scratch = (
        [pltpu.VMEM((nchunk, ch), jnp.int32) for _ in widths]
        + [pltpu.VMEM((ch, w), jnp.int32) for w in widths for _ in range(nb)]
        + [pltpu.SemaphoreType.DMA for _ in widths for _ in range(2 * nb)]
    )

    @functools.partial(pl.kernel, out_type=out_type, mesh=mesh,
                       scratch_types=scratch)
    def gather_kernel(*refs):
        tables = refs[0:nt]
        idxs = refs[nt:2 * nt]
        outs = refs[2 * nt:3 * nt]
        k = 3 * nt
        islabs = refs[k:k + nt]
        k += nt
        rbufs = [refs[k + t * nb:k + (t + 1) * nb] for t in range(nt)]
        k += nt * nb
        sems = [refs[k + t * nb:k + (t + 1) * nb] for t in range(nt)]
        k += nt * nb
        osems = [refs[k + t * nb:k + (t + 1) * nb] for t in range(nt)]
        wid = lax.axis_index("s") * _NC + lax.axis_index("c")
        base = pl.multiple_of(wid * ew, 8)
        crow = pl.multiple_of(wid * nchunk, 8)

        for t in range(nt):
            pltpu.sync_copy(idxs[t].at[pl.ds(crow, nchunk)], islabs[t])

        def fire(i, b):
            for t in range(nt):
                pltpu.async_copy(tables[t].at[islabs[t].at[i]], rbufs[t][b],
                                 sems[t][b])

        def drain(i, b):
            off = pl.multiple_of(base + i * ch, 8)
            for t in range(nt):
                pltpu.make_async_copy(tables[t].at[islabs[t].at[i]],
                                      rbufs[t][b], sems[t][b]).wait()
                pltpu.async_copy(rbufs[t][b], outs[t].at[pl.ds(off, ch)],
                                 osems[t][b])

        def wait_out(i, b):
            off = pl.multiple_of(base + i * ch, 8)
            for t in range(nt):
                pltpu.make_async_copy(rbufs[t][b],
                                      outs[t].at[pl.ds(off, ch)],
                                      osems[t][b]).wait()

        for i in range(nb):
            fire(i, i)

        def body(j, carry):
            i0 = j * nb
            for b in range(nb):
                i = i0 + b
                drain(i, b)

                @pl.when(i + nb < nchunk)
                def _next():
                    wait_out(i, b)
                    fire(i + nb, b)
            return carry

        lax.fori_loop(0, nchunk // nb, body, 0)
        for b in range(nb):
            wait_out(nchunk - nb + b, b)

    return gather_kernel


_EW2 = _EPAD // _NS
_NCH2 = _EW2 // _CHS


def _make_scatter():
    nb = 2
    mesh = plsc.VectorSubcoreMesh(core_axis_name="c", subcore_axis_name="s")
    out_type = [
        jax.ShapeDtypeStruct((_NPAD, _NDIM), _f32),
        jax.ShapeDtypeStruct((_NPAD, _NDIM), _f32),
    ]
    scratch = (
        [pltpu.VMEM_SHARED((_NPAD, _NDIM), _f32),
         pltpu.VMEM((_NCH2, _CHS), jnp.int32)]
        + [pltpu.VMEM((_CHS, _NDIM), _f32) for _ in range(nb)]
        + [pltpu.SemaphoreType.DMA for _ in range(2 * nb)]
    )

    @functools.partial(pl.kernel, out_type=out_type, mesh=mesh,
                       scratch_types=scratch)
    def scatter_kernel(w0_hbm, w1_hbm, e0_hbm, e1_hbm, dst_hbm, oz_hbm,
                       o_out, s_out,
                       acc, islab, wb0, wb1, ls0, ls1, as0, as1):
        c = lax.axis_index("c")
        s = lax.axis_index("s")
        wbs = [wb0, wb1]
        lsems = [ls0, ls1]
        asems = [as0, as1]

        @pl.when(s == 0)
        def _init():
            pltpu.sync_copy(oz_hbm, acc)

        crow = pl.multiple_of(s * _NCH2, 8)
        pltpu.sync_copy(dst_hbm.at[pl.ds(crow, _NCH2)], islab)
        plsc.subcore_barrier()

        def run(arr_hbm, sl):
            base = pl.multiple_of(sl * _EW2, 8)

            def fire_load(i, b):
                off = pl.multiple_of(base + i * _CHS, 8)
                pltpu.async_copy(arr_hbm.at[pl.ds(off, _CHS)], wbs[b],
                                 lsems[b])

            def step(i, b):
                off = pl.multiple_of(base + i * _CHS, 8)
                pltpu.make_async_copy(arr_hbm.at[pl.ds(off, _CHS)], wbs[b],
                                      lsems[b]).wait()
                pltpu.async_copy(wbs[b], acc.at[islab.at[i]], asems[b],
                                 add=True)

                @pl.when(i + nb < _NCH2)
                def _next():
                    pltpu.make_async_copy(wbs[b], acc.at[islab.at[i]],
                                          asems[b]).wait()
                    fire_load(i + nb, b)

            for i in range(nb):
                fire_load(i, i)

            def body(j, carry):
                for b in range(nb):
                    step(j * nb + b, b)
                return carry

            lax.fori_loop(0, _NCH2 // nb, body, 0)
            for b in range(nb):
                pltpu.make_async_copy(wbs[b], acc.at[islab.at[_NCH2 - nb + b]],
                                      asems[b]).wait()

        half = _NS // 2

        @pl.when((c == 0) & (s < half))
        def _scatter_w0():
            run(w0_hbm, s)

        @pl.when((c == 0) & (s >= half))
        def _scatter_w1():
            run(w1_hbm, s - half)

        @pl.when((c == 1) & (s < half))
        def _scatter_e0():
            run(e0_hbm, s)

        @pl.when((c == 1) & (s >= half))
        def _scatter_e1():
            run(e1_hbm, s - half)

        plsc.subcore_barrier()

        @pl.when((s == 0) & (c == 0))
        def _write_o():
            pltpu.sync_copy(acc, o_out)

        @pl.when((s == 0) & (c == 1))
        def _write_s():
            pltpu.sync_copy(acc, s_out)

    return scatter_kernel


_BN = 1000
_BE = 2048


def _bits16(x):
    y = x.astype(_bf16).astype(_f32)
    return lax.bitcast_convert_type(y, jnp.int32)


def _proj_body(x_ref, ws, bs, wd, bd, st_ref, dt_ref):
    x = x_ref[...]
    accs = jnp.dot(x, ws[...], preferred_element_type=_f32) + bs[...]
    accd = jnp.dot(x, wd[...], preferred_element_type=_f32) + bd[...]
    kp, vp, xsb = accs[:, :128], accs[:, 128:256], accs[:, 256:384]
    qp, xdb = accd[:, :128], accd[:, 128:256]
    hi_mask = jnp.int32(-65536)
    pk1 = (_bits16(vp) & hi_mask) | ((_bits16(kp) >> 16) & 0xFFFF)
    pk2 = (_bits16(xsb) >> 16) & 0xFFFF
    st_ref[...] = jnp.concatenate([pk1, pk2], axis=1)
    dt_ref[...] = (_bits16(xdb) & hi_mask) | ((_bits16(qp) >> 16) & 0xFFFF)


def _proj_call(x, ws, bs, wd, bd):
    grid = (_N // _BN,)
    full = lambda shape: pl.BlockSpec(shape, lambda i: (0,) * len(shape))
    rows = lambda w: pl.BlockSpec((_BN, w), lambda i: (i, 0))
    return pl.pallas_call(
        _proj_body,
        grid=grid,
        in_specs=[rows(_NDIM), full((_NDIM, _SW)), full((1, _SW)),
                  full((_NDIM, _DW)), full((1, _DW))],
        out_specs=[rows(_SWP), rows(_DWP)],
        out_shape=[jax.ShapeDtypeStruct((_N, _SWP), jnp.int32),
                   jax.ShapeDtypeStruct((_N, _DWP), jnp.int32)],
    )(x, ws, bs, wd, bd)


def _ln_mm(x, ones_mm, g, b):
    m = jnp.dot(x, ones_mm, preferred_element_type=_f32)
    v = jnp.dot(x * x, ones_mm, preferred_element_type=_f32) - m * m
    return (x - m) / jnp.sqrt(v + 1e-5) * g + b


def _unpack_lo(v):
    return lax.bitcast_convert_type(v << 16, _f32)


def _unpack_hi(v):
    return lax.bitcast_convert_type(v & jnp.int32(-65536), _f32)


def _edge_body(use_feat, row0, sg, dg, lg_in, rel,
               wlg, ba, ge1, be1, w1, b1, w2, b2, ge2, be2,
               w_ref, e128_ref, lgn_ref):
    r16 = lax.broadcasted_iota(jnp.int32, (_EDIM, _NDIM), 0)
    c16 = lax.broadcasted_iota(jnp.int32, (_EDIM, _NDIM), 1)
    t16 = (c16 % _EDIM == r16).astype(_f32)
    rs = lax.broadcasted_iota(jnp.int32, (_NDIM, _H), 0)
    cs = lax.broadcasted_iota(jnp.int32, (_NDIM, _H), 1)
    sel = (rs // _DH == cs).astype(_f32)
    rb = lax.broadcasted_iota(jnp.int32, (_H, _NDIM), 0)
    cb = lax.broadcasted_iota(jnp.int32, (_H, _NDIM), 1)
    selt = (cb // _DH == rb).astype(_f32)
    re = lax.broadcasted_iota(jnp.int32, (_NDIM, _EDIM), 0)
    ce = lax.broadcasted_iota(jnp.int32, (_NDIM, _EDIM), 1)
    ext16 = (re == ce).astype(_f32)

    s_i = sg[...]
    d_i = dg[...]
    ksv = s_i[:, :128]
    ks = _unpack_lo(ksv)
    vs = _unpack_hi(ksv)
    xsb = _unpack_lo(s_i[:, 128:256])
    qd = _unpack_lo(d_i)
    xdb = _unpack_hi(d_i)
    xss = jnp.dot(xsb, ext16, preferred_element_type=_f32)
    xdd = jnp.dot(xdb, ext16, preferred_element_type=_f32)

    if use_feat:
        lanes = lax.broadcasted_iota(jnp.int32, (_BE, _R), 1)
        onehot = (lg_in[...] == lanes).astype(_f32)
        lgb = jnp.dot(onehot, rel[...], preferred_element_type=_f32)
    else:
        lgb = lg_in[...]
    lg128 = jnp.dot(lgb, t16, preferred_element_type=_f32)
    t = qd * (ks + lg128)
    esc = jnp.exp(jnp.dot(t, sel, preferred_element_type=_f32) * 0.25)
    i = pl.program_id(0)
    rid = row0 + i * _BE + lax.broadcasted_iota(jnp.int32, (_BE, _H), 0)
    esc = jnp.where(rid < _E, esc, 0.0)
    e128 = jnp.dot(esc, selt, preferred_element_type=_f32)
    w_ref[...] = (vs + lg128) * e128
    e128_ref[...] = e128

    one16 = jnp.full((_EDIM, _EDIM), 1.0 / _EDIM, _f32)
    a = jnp.tanh(jnp.dot(lgb, wlg[...], preferred_element_type=_f32)
                 + xss + xdd + ba[...])
    eh = _ln_mm(lgb + a, one16, ge1[...], be1[...])
    ff = jnp.dot(jnp.maximum(jnp.dot(eh, w1[...], preferred_element_type=_f32)
                             + b1[...], 0.0),
                 w2[...], preferred_element_type=_f32) + b2[...]
    lgn_ref[...] = _ln_mm(eh + ff, one16, ge2[...], be2[...])


def _edge_call(use_feat, half, srcg, dstg, lg_or_feat, rel,
               wlg, ba, ge1, be1, w1, b1, w2, b2, ge2, be2):
    hpad = _EPAD // 2
    grid = (hpad // _BE,)
    full = lambda shape: pl.BlockSpec(shape, lambda i: (0,) * len(shape))
    rows = lambda w: pl.BlockSpec((_BE, w), lambda i: (i, 0))
    lg_spec = (pl.BlockSpec((_BE, 1), lambda i: (i, 0)) if use_feat
               else rows(_EDIM))
    return pl.pallas_call(
        functools.partial(_edge_body, use_feat, half * hpad),
        grid=grid,
        in_specs=[pl.BlockSpec((_BE, _SWP), lambda i: (i, 0)),
                  pl.BlockSpec((_BE, _DWP), lambda i: (i, 0)),
                  lg_spec,
                  full((_R, _EDIM)),
                  full((_EDIM, _EDIM)), full((1, _EDIM)),
                  full((1, _EDIM)), full((1, _EDIM)),
                  full((_EDIM, 4 * _EDIM)), full((1, 4 * _EDIM)),
                  full((4 * _EDIM, _EDIM)), full((1, _EDIM)),
                  full((1, _EDIM)), full((1, _EDIM))],
        out_specs=[rows(_NDIM), rows(_NDIM), rows(_EDIM)],
        out_shape=[jax.ShapeDtypeStruct((hpad, _NDIM), _f32),
                   jax.ShapeDtypeStruct((hpad, _NDIM), _f32),
                   jax.ShapeDtypeStruct((hpad, _EDIM), _f32)],
    )(srcg, dstg, lg_or_feat, rel,
      wlg, ba, ge1, be1, w1, b1, w2, b2, ge2, be2)


def _node_body(x_ref, op, sp, wo, bo, g1, b1, w1, f1, w2, f2, g2, b2,
               out_ref):
    o_un = op[...]
    sv = sp[...]
    o = o_un / (sv + 1e-12)
    o = jnp.dot(o, wo[...], preferred_element_type=_f32) + bo[...]
    one128 = jnp.full((_NDIM, _NDIM), 1.0 / _NDIM, _f32)
    h = _ln_mm(x_ref[...] + o, one128, g1[...], b1[...])
    ff = jnp.dot(jnp.maximum(jnp.dot(h, w1[...], preferred_element_type=_f32)
                             + f1[...], 0.0),
                 w2[...], preferred_element_type=_f32) + f2[...]
    out_ref[...] = _ln_mm(h + ff, one128, g2[...], b2[...])


def _node_call(x, o_part, s_part, wo, bo, g1, b1, w1, f1, w2, f2, g2, b2):
    grid = (_N // _BN,)
    full = lambda shape: pl.BlockSpec(shape, lambda i: (0,) * len(shape))
    return pl.pallas_call(
        _node_body,
        grid=grid,
        in_specs=[pl.BlockSpec((_BN, _NDIM), lambda i: (i, 0)),
                  pl.BlockSpec((_BN, _NDIM), lambda i: (i, 0)),
                  pl.BlockSpec((_BN, _NDIM), lambda i: (i, 0)),
                  full((_NDIM, _NDIM)), full((1, _NDIM)),
                  full((1, _NDIM)), full((1, _NDIM)),
                  full((_NDIM, 4 * _NDIM)), full((1, 4 * _NDIM)),
                  full((4 * _NDIM, _NDIM)), full((1, _NDIM)),
                  full((1, _NDIM)), full((1, _NDIM))],
        out_specs=pl.BlockSpec((_BN, _NDIM), lambda i: (i, 0)),
        out_shape=jax.ShapeDtypeStruct((_N, _NDIM), _f32),
    )(x, o_part, s_part, wo, bo, g1, b1, w1, f1, w2, f2, g2, b2)


def kernel(x, params, edge_feat, src_ids, dst_ids):
    p = params
    pad = _EPAD - _E
    hpad = _EPAD // 2
    src_p = jnp.pad(src_ids.astype(jnp.int32), (0, pad))
    dst_p = jnp.pad(dst_ids.astype(jnp.int32), (0, pad))
    feat_p = jnp.pad(edge_feat.astype(jnp.int32), (0, pad))
    feat_h = [feat_p[:hpad].reshape(hpad, 1), feat_p[hpad:].reshape(hpad, 1)]
    src_h = [src_p[:hpad].reshape(hpad // 64, 64),
             src_p[hpad:].reshape(hpad // 64, 64)]
    dst_h = [dst_p[:hpad].reshape(hpad // 64, 64),
             dst_p[hpad:].reshape(hpad // 64, 64)]
    dst_s = dst_p.reshape(_EPAD // _CHS, _CHS)

    gather = _make_gather()
    scatter = _make_scatter()
    o_zero = jnp.zeros((_NPAD, _NDIM), _f32)

    row = lambda v: v.reshape(1, -1)
    z16 = jnp.zeros((_NDIM, _EDIM), _f32)
    zpad = jnp.zeros((_NDIM, _SW - 2 * _NDIM - _EDIM), _f32)
    zpad_d = jnp.zeros((_NDIM, _DW - _NDIM - _EDIM), _f32)

    lg_h = [None, None]
    for l in range(_L):
        w_src = p['aff_W'][l][_EDIM:_EDIM + _NDIM]
        w_dst = p['aff_W'][l][_EDIM + _NDIM:]
        wlg = p['aff_W'][l][:_EDIM]
        ws = jnp.concatenate([p['Wk'][l], p['Wv'][l], w_src, zpad], axis=1)
        bs = jnp.concatenate([p['bk'][l], p['bv'][l],
                              jnp.zeros((_SW - 2 * _NDIM,), _f32)])
        wd = jnp.concatenate([p['Wq'][l], w_dst, zpad_d], axis=1)
        bd = jnp.concatenate([p['bq'][l], jnp.zeros((_DW - _NDIM,), _f32)])
        st, dt = _proj_call(x, ws, row(bs), wd, row(bd))
        use_feat = (l == 0)
        w_h, e_h, lgn_h = [], [], []
        for h in range(2):
            srcg, dstg = gather(st, dt, src_h[h], dst_h[h])
            w, e128, lgn = _edge_call(
                use_feat, h, srcg, dstg,
                feat_h[h] if use_feat else lg_h[h], p['rel_embed'],
                wlg, row(p['aff_b'][l]),
                row(p['ln_e1_g'][l]), row(p['ln_e1_b'][l]),
                p['ffn_e_W1'][l], row(p['ffn_e_b1'][l]),
                p['ffn_e_W2'][l], row(p['ffn_e_b2'][l]),
                row(p['ln_e2_g'][l]), row(p['ln_e2_b'][l]))
            w_h.append(w)
            e_h.append(e128)
            lgn_h.append(lgn)
        o_part, s_part = scatter(w_h[0], w_h[1], e_h[0], e_h[1], dst_s,
                                 o_zero)
        x = _node_call(
            x, o_part, s_part, p['Wo'][l], row(p['bo'][l]),
            row(p['ln_n1_g'][l]), row(p['ln_n1_b'][l]),
            p['ffn_n_W1'][l], row(p['ffn_n_b1'][l]),
            p['ffn_n_W2'][l], row(p['ffn_n_b2'][l]),
            row(p['ln_n2_g'][l]), row(p['ln_n2_b'][l]))
        lg_h = lgn_h
    return (x, jnp.concatenate(lg_h, axis=0)[:_E])


# speedup vs baseline: 27.5153x; 1.0020x over previous
import functools

import jax
import jax.numpy as jnp
from jax import lax
from jax.experimental import pallas as pl
from jax.experimental.pallas import tpu as pltpu
from jax.experimental.pallas import tpu_sc as plsc

_N = 10000
_E = 160000
_NDIM = 128
_H = 8
_DH = 16
_EDIM = 16
_L = 2
_R = 128

_NC = 2
_NS = 16
_NW = _NC * _NS

_EW = 5120
_EPAD = _EW * _NW

_SW = 384
_DW = 256
_SWP = 256
_DWP = 128

_NPAD = 10240
_CHS = 128

_f32 = jnp.float32
_bf16 = jnp.bfloat16


def _make_gather():
    epad, ch = _EPAD // 2, 64
    ew = epad // _NW
    nchunk = ew // ch
    widths = [_SWP, _DWP]
    nt = 2
    nb = 2
    mesh = plsc.VectorSubcoreMesh(core_axis_name="c", subcore_axis_name="s")
    out_type = [jax.ShapeDtypeStruct((epad, w), jnp.int32) for w in widths]
    scratch = (
        [pltpu.VMEM((nchunk, ch), jnp.int32) for _ in widths]
        + [pltpu.VMEM((ch, w), jnp.int32) for w in widths for _ in range(nb)]
        + [pltpu.SemaphoreType.DMA for _ in widths for _ in range(2 * nb)]
    )

    @functools.partial(pl.kernel, out_type=out_type, mesh=mesh,
                       scratch_types=scratch)
    def gather_kernel(*refs):
        tables = refs[0:nt]
        idxs = refs[nt:2 * nt]
        outs = refs[2 * nt:3 * nt]
        k = 3 * nt
        islabs = refs[k:k + nt]
        k += nt
        rbufs = [refs[k + t * nb:k + (t + 1) * nb] for t in range(nt)]
        k += nt * nb
        sems = [refs[k + t * nb:k + (t + 1) * nb] for t in range(nt)]
        k += nt * nb
        osems = [refs[k + t * nb:k + (t + 1) * nb] for t in range(nt)]
        wid = lax.axis_index("s") * _NC + lax.axis_index("c")
        base = pl.multiple_of(wid * ew, 8)
        crow = pl.multiple_of(wid * nchunk, 8)

        for t in range(nt):
            pltpu.sync_copy(idxs[t].at[pl.ds(crow, nchunk)], islabs[t])

        def fire(i, b):
            for t in range(nt):
                pltpu.async_copy(tables[t].at[islabs[t].at[i]], rbufs[t][b],
                                 sems[t][b])

        def drain(i, b):
            off = pl.multiple_of(base + i * ch, 8)
            for t in range(nt):
                pltpu.make_async_copy(tables[t].at[islabs[t].at[i]],
                                      rbufs[t][b], sems[t][b]).wait()
                pltpu.async_copy(rbufs[t][b], outs[t].at[pl.ds(off, ch)],
                                 osems[t][b])

        def wait_out(i, b):
            off = pl.multiple_of(base + i * ch, 8)
            for t in range(nt):
                pltpu.make_async_copy(rbufs[t][b],
                                      outs[t].at[pl.ds(off, ch)],
                                      osems[t][b]).wait()

        for i in range(nb):
            fire(i, i)

        def body(j, carry):
            i0 = j * nb
            for b in range(nb):
                i = i0 + b
                drain(i, b)

                @pl.when(i + nb < nchunk)
                def _next():
                    wait_out(i, b)
                    fire(i + nb, b)
            return carry

        lax.fori_loop(0, nchunk // nb, body, 0)
        for b in range(nb):
            wait_out(nchunk - nb + b, b)

    return gather_kernel


_EW2 = _EPAD // _NS
_NCH2 = _EW2 // _CHS


def _make_scatter():
    nb = 2
    mesh = plsc.VectorSubcoreMesh(core_axis_name="c", subcore_axis_name="s")
    out_type = [
        jax.ShapeDtypeStruct((_NPAD, _NDIM), _f32),
        jax.ShapeDtypeStruct((_NPAD, _NDIM), _f32),
    ]
    scratch = (
        [pltpu.VMEM_SHARED((_NPAD, _NDIM), _f32),
         pltpu.VMEM((_NCH2, _CHS), jnp.int32)]
        + [pltpu.VMEM((_CHS, _NDIM), _f32) for _ in range(nb)]
        + [pltpu.SemaphoreType.DMA for _ in range(nb)]
    )

    @functools.partial(pl.kernel, out_type=out_type, mesh=mesh,
                       scratch_types=scratch)
    def scatter_kernel(w0_hbm, w1_hbm, e0_hbm, e1_hbm, dst_hbm, oz_hbm,
                       o_out, s_out,
                       acc, islab, wb0, wb1, ls0, ls1):
        c = lax.axis_index("c")
        s = lax.axis_index("s")
        wbs = [wb0, wb1]
        lsems = [ls0, ls1]

        @pl.when(s == 0)
        def _init():
            pltpu.sync_copy(oz_hbm, acc)

        crow = pl.multiple_of(s * _NCH2, 8)
        pltpu.sync_copy(dst_hbm.at[pl.ds(crow, _NCH2)], islab)
        plsc.subcore_barrier()

        def run(arr_hbm, sl):
            base = pl.multiple_of(sl * _EW2, 8)

            def fire_load(i, b):
                off = pl.multiple_of(base + i * _CHS, 8)
                pltpu.async_copy(arr_hbm.at[pl.ds(off, _CHS)], wbs[b],
                                 lsems[b])

            def step(i, b):
                off = pl.multiple_of(base + i * _CHS, 8)
                pltpu.make_async_copy(arr_hbm.at[pl.ds(off, _CHS)], wbs[b],
                                      lsems[b]).wait()
                pltpu.sync_copy(wbs[b], acc.at[islab.at[i]], add=True)

                @pl.when(i + nb < _NCH2)
                def _next():
                    fire_load(i + nb, b)

            for i in range(nb):
                fire_load(i, i)

            def body(j, carry):
                for b in range(nb):
                    step(j * nb + b, b)
                return carry

            lax.fori_loop(0, _NCH2 // nb, body, 0)

        half = _NS // 2

        @pl.when((c == 0) & (s < half))
        def _scatter_w0():
            run(w0_hbm, s)

        @pl.when((c == 0) & (s >= half))
        def _scatter_w1():
            run(w1_hbm, s - half)

        @pl.when((c == 1) & (s < half))
        def _scatter_e0():
            run(e0_hbm, s)

        @pl.when((c == 1) & (s >= half))
        def _scatter_e1():
            run(e1_hbm, s - half)

        plsc.subcore_barrier()

        @pl.when((s == 0) & (c == 0))
        def _write_o():
            pltpu.sync_copy(acc, o_out)

        @pl.when((s == 0) & (c == 1))
        def _write_s():
            pltpu.sync_copy(acc, s_out)

    return scatter_kernel


_BN = 1000
_BE = 2048


def _bits16(x):
    y = x.astype(_bf16).astype(_f32)
    return lax.bitcast_convert_type(y, jnp.int32)


def _proj_body(x_ref, ws, bs, wd, bd, st_ref, dt_ref):
    x = x_ref[...]
    accs = jnp.dot(x, ws[...], preferred_element_type=_f32) + bs[...]
    accd = jnp.dot(x, wd[...], preferred_element_type=_f32) + bd[...]
    kp, vp, xsb = accs[:, :128], accs[:, 128:256], accs[:, 256:384]
    qp, xdb = accd[:, :128], accd[:, 128:256]
    hi_mask = jnp.int32(-65536)
    pk1 = (_bits16(vp) & hi_mask) | ((_bits16(kp) >> 16) & 0xFFFF)
    pk2 = (_bits16(xsb) >> 16) & 0xFFFF
    st_ref[...] = jnp.concatenate([pk1, pk2], axis=1)
    dt_ref[...] = (_bits16(xdb) & hi_mask) | ((_bits16(qp) >> 16) & 0xFFFF)


def _proj_call(x, ws, bs, wd, bd):
    grid = (_N // _BN,)
    full = lambda shape: pl.BlockSpec(shape, lambda i: (0,) * len(shape))
    rows = lambda w: pl.BlockSpec((_BN, w), lambda i: (i, 0))
    return pl.pallas_call(
        _proj_body,
        grid=grid,
        in_specs=[rows(_NDIM), full((_NDIM, _SW)), full((1, _SW)),
                  full((_NDIM, _DW)), full((1, _DW))],
        out_specs=[rows(_SWP), rows(_DWP)],
        out_shape=[jax.ShapeDtypeStruct((_N, _SWP), jnp.int32),
                   jax.ShapeDtypeStruct((_N, _DWP), jnp.int32)],
    )(x, ws, bs, wd, bd)


def _ln_mm(x, ones_mm, g, b):
    m = jnp.dot(x, ones_mm, preferred_element_type=_f32)
    v = jnp.dot(x * x, ones_mm, preferred_element_type=_f32) - m * m
    return (x - m) / jnp.sqrt(v + 1e-5) * g + b


def _unpack_lo(v):
    return lax.bitcast_convert_type(v << 16, _f32)


def _unpack_hi(v):
    return lax.bitcast_convert_type(v & jnp.int32(-65536), _f32)


def _edge_body(use_feat, row0, sg, dg, lg_in, rel,
               wlg, ba, ge1, be1, w1, b1, w2, b2, ge2, be2,
               w_ref, e128_ref, lgn_ref):
    r16 = lax.broadcasted_iota(jnp.int32, (_EDIM, _NDIM), 0)
    c16 = lax.broadcasted_iota(jnp.int32, (_EDIM, _NDIM), 1)
    t16 = (c16 % _EDIM == r16).astype(_f32)
    rs = lax.broadcasted_iota(jnp.int32, (_NDIM, _H), 0)
    cs = lax.broadcasted_iota(jnp.int32, (_NDIM, _H), 1)
    sel = (rs // _DH == cs).astype(_f32)
    rb = lax.broadcasted_iota(jnp.int32, (_H, _NDIM), 0)
    cb = lax.broadcasted_iota(jnp.int32, (_H, _NDIM), 1)
    selt = (cb // _DH == rb).astype(_f32)
    re = lax.broadcasted_iota(jnp.int32, (_NDIM, _EDIM), 0)
    ce = lax.broadcasted_iota(jnp.int32, (_NDIM, _EDIM), 1)
    ext16 = (re == ce).astype(_f32)

    s_i = sg[...]
    d_i = dg[...]
    ksv = s_i[:, :128]
    ks = _unpack_lo(ksv)
    vs = _unpack_hi(ksv)
    xsb = _unpack_lo(s_i[:, 128:256])
    qd = _unpack_lo(d_i)
    xdb = _unpack_hi(d_i)
    xss = jnp.dot(xsb, ext16, preferred_element_type=_f32)
    xdd = jnp.dot(xdb, ext16, preferred_element_type=_f32)

    if use_feat:
        lanes = lax.broadcasted_iota(jnp.int32, (_BE, _R), 1)
        onehot = (lg_in[...] == lanes).astype(_f32)
        lgb = jnp.dot(onehot, rel[...], preferred_element_type=_f32)
    else:
        lgb = lg_in[...]
    lg128 = jnp.dot(lgb, t16, preferred_element_type=_f32)
    t = qd * (ks + lg128)
    esc = jnp.exp(jnp.dot(t, sel, preferred_element_type=_f32) * 0.25)
    i = pl.program_id(0)
    rid = row0 + i * _BE + lax.broadcasted_iota(jnp.int32, (_BE, _H), 0)
    esc = jnp.where(rid < _E, esc, 0.0)
    e128 = jnp.dot(esc, selt, preferred_element_type=_f32)
    w_ref[...] = (vs + lg128) * e128
    e128_ref[...] = e128

    one16 = jnp.full((_EDIM, _EDIM), 1.0 / _EDIM, _f32)
    a = jnp.tanh(jnp.dot(lgb, wlg[...], preferred_element_type=_f32)
                 + xss + xdd + ba[...])
    eh = _ln_mm(lgb + a, one16, ge1[...], be1[...])
    ff = jnp.dot(jnp.maximum(jnp.dot(eh, w1[...], preferred_element_type=_f32)
                             + b1[...], 0.0),
                 w2[...], preferred_element_type=_f32) + b2[...]
    lgn_ref[...] = _ln_mm(eh + ff, one16, ge2[...], be2[...])


def _edge_call(use_feat, half, srcg, dstg, lg_or_feat, rel,
               wlg, ba, ge1, be1, w1, b1, w2, b2, ge2, be2):
    hpad = _EPAD // 2
    grid = (hpad // _BE,)
    full = lambda shape: pl.BlockSpec(shape, lambda i: (0,) * len(shape))
    rows = lambda w: pl.BlockSpec((_BE, w), lambda i: (i, 0))
    lg_spec = (pl.BlockSpec((_BE, 1), lambda i: (i, 0)) if use_feat
               else rows(_EDIM))
    return pl.pallas_call(
        functools.partial(_edge_body, use_feat, half * hpad),
        grid=grid,
        in_specs=[pl.BlockSpec((_BE, _SWP), lambda i: (i, 0)),
                  pl.BlockSpec((_BE, _DWP), lambda i: (i, 0)),
                  lg_spec,
                  full((_R, _EDIM)),
                  full((_EDIM, _EDIM)), full((1, _EDIM)),
                  full((1, _EDIM)), full((1, _EDIM)),
                  full((_EDIM, 4 * _EDIM)), full((1, 4 * _EDIM)),
                  full((4 * _EDIM, _EDIM)), full((1, _EDIM)),
                  full((1, _EDIM)), full((1, _EDIM))],
        out_specs=[rows(_NDIM), rows(_NDIM), rows(_EDIM)],
        out_shape=[jax.ShapeDtypeStruct((hpad, _NDIM), _f32),
                   jax.ShapeDtypeStruct((hpad, _NDIM), _f32),
                   jax.ShapeDtypeStruct((hpad, _EDIM), _f32)],
    )(srcg, dstg, lg_or_feat, rel,
      wlg, ba, ge1, be1, w1, b1, w2, b2, ge2, be2)


def _node_body(x_ref, op, sp, wo, bo, g1, b1, w1, f1, w2, f2, g2, b2,
               out_ref):
    o_un = op[...]
    sv = sp[...]
    o = o_un / (sv + 1e-12)
    o = jnp.dot(o, wo[...], preferred_element_type=_f32) + bo[...]
    one128 = jnp.full((_NDIM, _NDIM), 1.0 / _NDIM, _f32)
    h = _ln_mm(x_ref[...] + o, one128, g1[...], b1[...])
    ff = jnp.dot(jnp.maximum(jnp.dot(h, w1[...], preferred_element_type=_f32)
                             + f1[...], 0.0),
                 w2[...], preferred_element_type=_f32) + f2[...]
    out_ref[...] = _ln_mm(h + ff, one128, g2[...], b2[...])


def _node_call(x, o_part, s_part, wo, bo, g1, b1, w1, f1, w2, f2, g2, b2):
    grid = (_N // _BN,)
    full = lambda shape: pl.BlockSpec(shape, lambda i: (0,) * len(shape))
    return pl.pallas_call(
        _node_body,
        grid=grid,
        in_specs=[pl.BlockSpec((_BN, _NDIM), lambda i: (i, 0)),
                  pl.BlockSpec((_BN, _NDIM), lambda i: (i, 0)),
                  pl.BlockSpec((_BN, _NDIM), lambda i: (i, 0)),
                  full((_NDIM, _NDIM)), full((1, _NDIM)),
                  full((1, _NDIM)), full((1, _NDIM)),
                  full((_NDIM, 4 * _NDIM)), full((1, 4 * _NDIM)),
                  full((4 * _NDIM, _NDIM)), full((1, _NDIM)),
                  full((1, _NDIM)), full((1, _NDIM))],
        out_specs=pl.BlockSpec((_BN, _NDIM), lambda i: (i, 0)),
        out_shape=jax.ShapeDtypeStruct((_N, _NDIM), _f32),
    )(x, o_part, s_part, wo, bo, g1, b1, w1, f1, w2, f2, g2, b2)


def kernel(x, params, edge_feat, src_ids, dst_ids):
    p = params
    pad = _EPAD - _E
    hpad = _EPAD // 2
    src_p = jnp.pad(src_ids.astype(jnp.int32), (0, pad))
    dst_p = jnp.pad(dst_ids.astype(jnp.int32), (0, pad))
    feat_p = jnp.pad(edge_feat.astype(jnp.int32), (0, pad))
    feat_h = [feat_p[:hpad].reshape(hpad, 1), feat_p[hpad:].reshape(hpad, 1)]
    src_h = [src_p[:hpad].reshape(hpad // 64, 64),
             src_p[hpad:].reshape(hpad // 64, 64)]
    dst_h = [dst_p[:hpad].reshape(hpad // 64, 64),
             dst_p[hpad:].reshape(hpad // 64, 64)]
    dst_s = dst_p.reshape(_EPAD // _CHS, _CHS)

    gather = _make_gather()
    scatter = _make_scatter()
    o_zero = jnp.zeros((_NPAD, _NDIM), _f32)

    row = lambda v: v.reshape(1, -1)
    z16 = jnp.zeros((_NDIM, _EDIM), _f32)
    zpad = jnp.zeros((_NDIM, _SW - 2 * _NDIM - _EDIM), _f32)
    zpad_d = jnp.zeros((_NDIM, _DW - _NDIM - _EDIM), _f32)

    lg_h = [None, None]
    for l in range(_L):
        w_src = p['aff_W'][l][_EDIM:_EDIM + _NDIM]
        w_dst = p['aff_W'][l][_EDIM + _NDIM:]
        wlg = p['aff_W'][l][:_EDIM]
        ws = jnp.concatenate([p['Wk'][l], p['Wv'][l], w_src, zpad], axis=1)
        bs = jnp.concatenate([p['bk'][l], p['bv'][l],
                              jnp.zeros((_SW - 2 * _NDIM,), _f32)])
        wd = jnp.concatenate([p['Wq'][l], w_dst, zpad_d], axis=1)
        bd = jnp.concatenate([p['bq'][l], jnp.zeros((_DW - _NDIM,), _f32)])
        st, dt = _proj_call(x, ws, row(bs), wd, row(bd))
        use_feat = (l == 0)
        w_h, e_h, lgn_h = [], [], []
        for h in range(2):
            srcg, dstg = gather(st, dt, src_h[h], dst_h[h])
            w, e128, lgn = _edge_call(
                use_feat, h, srcg, dstg,
                feat_h[h] if use_feat else lg_h[h], p['rel_embed'],
                wlg, row(p['aff_b'][l]),
                row(p['ln_e1_g'][l]), row(p['ln_e1_b'][l]),
                p['ffn_e_W1'][l], row(p['ffn_e_b1'][l]),
                p['ffn_e_W2'][l], row(p['ffn_e_b2'][l]),
                row(p['ln_e2_g'][l]), row(p['ln_e2_b'][l]))
            w_h.append(w)
            e_h.append(e128)
            lgn_h.append(lgn)
        o_part, s_part = scatter(w_h[0], w_h[1], e_h[0], e_h[1], dst_s,
                                 o_zero)
        x = _node_call(
            x, o_part, s_part, p['Wo'][l], row(p['bo'][l]),
            row(p['ln_n1_g'][l]), row(p['ln_n1_b'][l]),
            p['ffn_n_W1'][l], row(p['ffn_n_b1'][l]),
            p['ffn_n_W2'][l], row(p['ffn_n_b2'][l]),
            row(p['ln_n2_g'][l]), row(p['ln_n2_b'][l]))
        lg_h = lgn_h
    return (x, jnp.concatenate(lg_h, axis=0)[:_E])
